# Initial kernel scaffold; baseline (speedup 1.0000x reference)
import jax
import jax.numpy as jnp
from jax.experimental import pallas as pl


def kernel(H, Z, block_id, batch_id, edges, edge_attr, Wq, Wk, Wv, We, Wo, Wc):
    raise NotImplementedError("write your pallas kernel here")


# speedup vs baseline: 1.1487x; 1.1487x over previous
import functools

import jax
import jax.numpy as jnp
import numpy as np
from jax import lax
from jax.experimental import pallas as pl
from jax.experimental.pallas import tpu as pltpu
from jax.experimental.pallas import tpu_sc as plsc

N_UNIT = 100000
N_BLOCK = 25000
N_GRAPH = 64
E = 400000
HIDDEN = 128
NRBF = 16
EDGE = 16
NHEAD = 4
DH = HIDDEN // NHEAD
LAYERS = 3
CUTOFF = 7.0

NC, NS = 2, 16
NW = NC * NS
SP = 25088
EP = 409600
NUP = 102400
SGP = 128
DS = 16
D_DST = 272
D_SRC = 144

_mesh = lambda: plsc.VectorSubcoreMesh(core_axis_name="c", subcore_axis_name="s")


def _sc_gather2(tab1, idx1, tab2, idx2, d1, d2, ch=128):
    n = idx1.shape[0]
    per_w = n // NW
    iters = per_w // ch

    @functools.partial(
        pl.kernel, mesh=_mesh(),
        compiler_params=pltpu.CompilerParams(use_tc_tiling_on_sc=False),
        out_type=(jax.ShapeDtypeStruct((n, d1), jnp.float32),
                  jax.ShapeDtypeStruct((n, d2), jnp.float32)),
        scratch_types=[pltpu.VMEM((ch,), jnp.int32),
                       pltpu.VMEM((ch,), jnp.int32),
                       pltpu.VMEM((ch, d1), jnp.float32),
                       pltpu.VMEM((ch, d2), jnp.float32),
                       pltpu.SemaphoreType.DMA,
                       pltpu.SemaphoreType.DMA],
    )
    def k(t1, i1, t2, i2, o1, o2, iv1, iv2, r1, r2, s1, s2):
        wid = lax.axis_index("s") * NC + lax.axis_index("c")
        base = wid * per_w

        def body(i, carry):
            b = base + i * ch
            pltpu.sync_copy(i1.at[pl.ds(b, ch)], iv1)
            pltpu.sync_copy(i2.at[pl.ds(b, ch)], iv2)
            c1 = pltpu.async_copy(t1.at[iv1], r1, s1)
            c2 = pltpu.async_copy(t2.at[iv2], r2, s2)
            c1.wait()
            c2.wait()
            pltpu.sync_copy(r1, o1.at[pl.ds(b, ch)])
            pltpu.sync_copy(r2, o2.at[pl.ds(b, ch)])
            return carry

        lax.fori_loop(0, iters, body, 0)

    return k(tab1, idx1, tab2, idx2)


def _sc_gather1(tab, idx, d, ch=128):
    n = idx.shape[0]
    per_w = n // NW
    iters = per_w // ch

    @functools.partial(
        pl.kernel, mesh=_mesh(),
        compiler_params=pltpu.CompilerParams(use_tc_tiling_on_sc=False),
        out_type=jax.ShapeDtypeStruct((n, d), jnp.float32),
        scratch_types=[pltpu.VMEM((ch,), jnp.int32),
                       pltpu.VMEM((ch, d), jnp.float32),
                       pltpu.SemaphoreType.DMA],
    )
    def k(t1, i1, o1, iv1, r1, s1):
        wid = lax.axis_index("s") * NC + lax.axis_index("c")
        base = wid * per_w

        def body(i, carry):
            b = base + i * ch
            pltpu.sync_copy(i1.at[pl.ds(b, ch)], iv1)
            pltpu.async_copy(t1.at[iv1], r1, s1).wait()
            pltpu.sync_copy(r1, o1.at[pl.ds(b, ch)])
            return carry

        lax.fori_loop(0, iters, body, 0)

    return k(tab, idx)


def _sc_scatter_cols(vals3, idx, s_out, zeros64, ch=128):
    n = idx.shape[0]
    per_t = n // NS
    iters = per_t // ch
    stripe = s_out // NS

    @functools.partial(
        pl.kernel, mesh=_mesh(),
        compiler_params=pltpu.CompilerParams(use_tc_tiling_on_sc=False),
        out_type=jax.ShapeDtypeStruct((NC, s_out, 64), jnp.float32),
        scratch_types=[pltpu.VMEM((ch,), jnp.int32),
                       pltpu.VMEM((ch, 64), jnp.float32),
                       pltpu.VMEM_SHARED((s_out, 64), jnp.float32)],
    )
    def k(v_hbm, i_hbm, z_hbm, o_hbm, iv, vv, acc):
        cid = lax.axis_index("c")
        sid = lax.axis_index("s")
        r0 = sid * stripe
        pltpu.sync_copy(z_hbm.at[pl.ds(r0, stripe)], acc.at[pl.ds(r0, stripe)])
        plsc.subcore_barrier()

        def body(i, carry):
            b = sid * per_t + i * ch
            pltpu.sync_copy(i_hbm.at[pl.ds(b, ch)], iv)
            pltpu.sync_copy(v_hbm.at[cid, pl.ds(b, ch), :], vv)
            pltpu.sync_copy(vv, acc.at[iv], add=True)
            return carry

        lax.fori_loop(0, iters, body, 0)
        plsc.subcore_barrier()
        pltpu.sync_copy(acc.at[pl.ds(r0, stripe)],
                        o_hbm.at[cid, pl.ds(r0, stripe), :])

    return k(vals3, idx, zeros64)


def _sc_scatter_small(vals, idx, s_out, zeros16, ch=128):
    n = idx.shape[0]
    per_w = n // NW
    iters = per_w // ch
    stripe = s_out // NS

    @functools.partial(
        pl.kernel, mesh=_mesh(),
        compiler_params=pltpu.CompilerParams(use_tc_tiling_on_sc=False),
        out_type=jax.ShapeDtypeStruct((NC, s_out, DS), jnp.float32),
        scratch_types=[pltpu.VMEM((ch,), jnp.int32),
                       pltpu.VMEM((ch, DS), jnp.float32),
                       pltpu.VMEM_SHARED((s_out, DS), jnp.float32)],
    )
    def k(v_hbm, i_hbm, z_hbm, o_hbm, iv, vv, acc):
        cid = lax.axis_index("c")
        sid = lax.axis_index("s")
        wid = sid * NC + cid
        r0 = sid * stripe
        pltpu.sync_copy(z_hbm.at[pl.ds(r0, stripe)], acc.at[pl.ds(r0, stripe)])
        plsc.subcore_barrier()

        def body(i, carry):
            b = wid * per_w + i * ch
            pltpu.sync_copy(i_hbm.at[pl.ds(b, ch)], iv)
            pltpu.sync_copy(v_hbm.at[pl.ds(b, ch), :], vv)
            pltpu.sync_copy(vv, acc.at[iv], add=True)
            return carry

        lax.fori_loop(0, iters, body, 0)
        plsc.subcore_barrier()
        pltpu.sync_copy(acc.at[pl.ds(r0, stripe)],
                        o_hbm.at[cid, pl.ds(r0, stripe), :])

    return k(vals, idx, zeros16)


def _tc(body, grid, in_specs, out_shapes, out_specs):
    return pl.pallas_call(
        body, grid=grid, in_specs=in_specs,
        out_shape=out_shapes, out_specs=out_specs)


def _full(shape):
    return pl.BlockSpec(shape, lambda i: tuple(0 for _ in shape))


def _rows(shape):
    nd = len(shape)
    return pl.BlockSpec(shape, lambda i: (i,) + (0,) * (nd - 1))


def _mid(shape):
    return pl.BlockSpec(shape, lambda i: (0, i, 0))


SB = 784
RB = 2048


def _tables(hb, zc, wq, wk, wv, wet, wc, tdst_ref, tsrca_ref, tsrcc_ref):
    q = jnp.dot(hb, wq, preferred_element_type=jnp.float32)
    kk = jnp.dot(hb, wk, preferred_element_type=jnp.float32)
    v = jnp.dot(hb, wv, preferred_element_type=jnp.float32)
    a = jnp.concatenate(
        [jnp.dot(q[:, 32 * h:32 * h + 32], wet[32 * h:32 * h + 32, :],
                 preferred_element_type=jnp.float32) for h in range(NHEAD)],
        axis=1)
    vc = jnp.sum((v * wc[:, 0][None, :]).reshape(-1, NHEAD, DH), axis=-1)
    z3 = zc[:, 0:3]
    zpad = jnp.zeros((hb.shape[0], D_DST - 259), jnp.float32)
    tdst_ref[...] = jnp.concatenate([q, a, z3, zpad], axis=1)
    spad = jnp.zeros((hb.shape[0], D_SRC - 131), jnp.float32)
    tsrca_ref[...] = jnp.concatenate([kk, z3, spad], axis=1)
    cpad = jnp.zeros((hb.shape[0], D_SRC - 135), jnp.float32)
    tsrcc_ref[...] = jnp.concatenate([v, vc, z3, cpad], axis=1)


def _k_pool0_body(hs_ref, zcnt_ref, degp_ref, wq_ref, wk_ref, wv_ref,
                  wet_ref, wc_ref, hb_ref, zc_ref, degv_ref,
                  tdst_ref, tsrca_ref, tsrcc_ref):
    zc01 = zcnt_ref[0] + zcnt_ref[1]
    cnt = jnp.maximum(zc01[:, 3:4], 1.0)
    hb = jnp.concatenate([hs_ref[0], hs_ref[1]], axis=1) / cnt
    z3 = zc01[:, 0:3] / cnt
    deg01 = degp_ref[0] + degp_ref[1]
    deg = jnp.maximum(deg01[:, 0:1], 1.0)
    pad13 = jnp.zeros((SB, DS - 3), jnp.float32)
    zc = jnp.concatenate([z3, pad13], axis=1)
    hb_ref[...] = hb
    zc_ref[...] = zc
    pad15 = jnp.zeros((SB, DS - 1), jnp.float32)
    degv_ref[...] = jnp.concatenate([deg, pad15], axis=1)
    _tables(hb, zc, wq_ref[...], wk_ref[...], wv_ref[...], wet_ref[...],
            wc_ref[...], tdst_ref, tsrca_ref, tsrcc_ref)


def _k_upd_body(hc_ref, zc_ref, agg_ref, dzp_ref, degv_ref, wo_ref,
                wq_ref, wk_ref, wv_ref, wet_ref, wc_ref,
                hcn_ref, zcn_ref, tdst_ref, tsrca_ref, tsrcc_ref):
    agg = jnp.concatenate([agg_ref[0], agg_ref[1]], axis=1)
    hc = hc_ref[...] + jnp.dot(agg, wo_ref[...],
                               preferred_element_type=jnp.float32)
    dz01 = dzp_ref[0] + dzp_ref[1]
    deg = degv_ref[...][:, 0:1]
    z3 = zc_ref[...][:, 0:3] + dz01[:, 0:3] / deg
    pad13 = jnp.zeros((SB, DS - 3), jnp.float32)
    zc = jnp.concatenate([z3, pad13], axis=1)
    hcn_ref[...] = hc
    zcn_ref[...] = zc
    _tables(hc, zc, wq_ref[...], wk_ref[...], wv_ref[...], wet_ref[...],
            wc_ref[...], tdst_ref, tsrca_ref, tsrcc_ref)


_RBF_DEN = 2.0 * (CUTOFF / NRBF) ** 2
_ISQ = 1.0 / np.sqrt(DH)


def _k_logits_body(gd_ref, gs_ref, ea_ref, lo_ref):
    i = pl.program_id(0)
    gd = gd_ref[...]
    gs = gs_ref[...]
    rel = gd[:, 256:259] - gs[:, 128:131]
    d = jnp.sqrt(jnp.sum(rel * rel, axis=1, keepdims=True) + 1e-8)
    centers = (lax.broadcasted_iota(jnp.int32, (1, NRBF), 1).astype(jnp.float32)
               * (CUTOFF / (NRBF - 1)))
    rbf = jnp.exp(-((d - centers) ** 2) / _RBF_DEN)
    feat = jnp.concatenate([rbf, ea_ref[...]], axis=1)
    qk = jnp.sum((gd[:, 0:128] * gs[:, 0:128]).reshape(RB, NHEAD, DH), axis=-1)
    fa = jnp.sum(gd[:, 128:256].reshape(RB, NHEAD, DH) * feat[:, None, :],
                 axis=-1)
    logits = (qk + fa) * _ISQ
    rows = i * RB + lax.broadcasted_iota(jnp.int32, (RB, 1), 0)
    logits = jnp.where(rows < E, logits, 0.0)
    lo_ref[...] = jnp.concatenate(
        [logits, jnp.zeros((RB, DS - NHEAD), jnp.float32)], axis=1)


def _k_ctab_body(lsum_ref, degv_ref, ct_ref):
    lsum = lsum_ref[0] + lsum_ref[1]
    c = lsum[:, 0:NHEAD] / degv_ref[...][:, 0:1]
    ct_ref[...] = jnp.concatenate(
        [c, jnp.zeros((SB, DS - NHEAD), jnp.float32)], axis=1)


def _k_w_body(lo_ref, gc_ref, w_ref):
    i = pl.program_id(0)
    w = jnp.exp(lo_ref[...][:, 0:NHEAD] - gc_ref[...][:, 0:NHEAD])
    rows = i * RB + lax.broadcasted_iota(jnp.int32, (RB, 1), 0)
    w = jnp.where(rows < E, w, 0.0)
    w_ref[...] = jnp.concatenate(
        [w, jnp.zeros((RB, DS - NHEAD), jnp.float32)], axis=1)


def _k_ddst_body(denp_ref, zc_ref, dd_ref):
    den = denp_ref[0] + denp_ref[1]
    dd = den[:, 0:NHEAD] + 1e-8
    z3 = zc_ref[...][:, 0:3]
    dd_ref[...] = jnp.concatenate(
        [dd, z3, jnp.zeros((SB, DS - NHEAD - 3), jnp.float32)], axis=1)


def _k_msg_body(w_ref, gdd_ref, gs_ref, msg_ref, dzv_ref):
    gdd = gdd_ref[...]
    gs = gs_ref[...]
    alpha = w_ref[...][:, 0:NHEAD] / gdd[:, 0:NHEAD]
    v = gs[:, 0:128].reshape(RB, NHEAD, DH)
    msg = (alpha[:, :, None] * v).reshape(RB, HIDDEN)
    msg_ref[...] = jnp.stack([msg[:, 0:64], msg[:, 64:128]], axis=0)
    csc = jnp.sum(alpha * gs[:, 128:132], axis=1, keepdims=True)
    rel = gdd[:, NHEAD:NHEAD + 3] - gs[:, 132:135]
    dzv_ref[...] = jnp.concatenate(
        [rel * csc, jnp.zeros((RB, DS - 3), jnp.float32)], axis=1)


def _k_final_body(hc_ref, agg_ref, wo_ref, br_ref, br3_ref):
    agg = jnp.concatenate([agg_ref[0], agg_ref[1]], axis=1)
    hc = hc_ref[...] + jnp.dot(agg, wo_ref[...],
                               preferred_element_type=jnp.float32)
    nrm = jnp.sqrt(jnp.sum(hc * hc, axis=1, keepdims=True))
    br = hc / (nrm + 1e-12)
    br_ref[...] = br
    br3_ref[...] = jnp.stack([br[:, 0:64], br[:, 64:128]], axis=0)


def _k_graph_body(grp_ref, gr_ref):
    g = jnp.concatenate([grp_ref[0], grp_ref[1]], axis=1)
    nrm = jnp.sqrt(jnp.sum(g * g, axis=1, keepdims=True))
    gr_ref[...] = g / (nrm + 1e-12)


def kernel(H, Z, block_id, batch_id, edges, edge_attr, Wq, Wk, Wv, We, Wo, Wc):
    f32 = jnp.float32
    src = jnp.pad(edges[0].astype(jnp.int32), (0, EP - E))
    dst = jnp.pad(edges[1].astype(jnp.int32), (0, EP - E))
    ea_p = jnp.pad(edge_attr, ((0, EP - E), (0, 0)))
    emask = (jnp.arange(EP) < E).astype(f32)[:, None]
    degvals = jnp.pad(emask, ((0, 0), (0, DS - 1)))
    bid_p = jnp.pad(block_id.astype(jnp.int32), (0, NUP - N_UNIT))
    hp = jnp.pad(H, ((0, NUP - N_UNIT), (0, 0)))
    h3 = jnp.stack([hp[:, 0:64], hp[:, 64:128]], axis=0)
    zuv = jnp.pad(
        jnp.concatenate([Z.reshape(N_UNIT, 3), jnp.ones((N_UNIT, 1), f32)],
                        axis=1),
        ((0, NUP - N_UNIT), (0, DS - 4)))
    batch_p = jnp.pad(batch_id.astype(jnp.int32), (0, SP - N_BLOCK))
    z64 = jnp.zeros((SP, 64), f32)
    z64g = jnp.zeros((SGP, 64), f32)
    z16 = jnp.zeros((SP, DS), f32)
    wet = jnp.transpose(We, (0, 2, 1))

    hsum3 = _sc_scatter_cols(h3, bid_p, SP, z64)
    zcnt = _sc_scatter_small(zuv, bid_p, SP, z16)
    degp = _sc_scatter_small(degvals, dst, SP, z16)

    gs = (32,)
    w_spec = _full((HIDDEN, HIDDEN))
    wet_spec = _full((HIDDEN, 32))
    wc_spec = _full((HIDDEN, 1))
    tbl_out = (jax.ShapeDtypeStruct((SP, D_DST), f32),
               jax.ShapeDtypeStruct((SP, D_SRC), f32),
               jax.ShapeDtypeStruct((SP, D_SRC), f32))
    tbl_spec = (_rows((SB, D_DST)), _rows((SB, D_SRC)), _rows((SB, D_SRC)))

    hb, zc, degv, tdst, tsrca, tsrcc = _tc(
        _k_pool0_body, gs,
        [_mid((NC, SB, 64)), _mid((NC, SB, DS)), _mid((NC, SB, DS)),
         w_spec, w_spec, w_spec, wet_spec, wc_spec],
        (jax.ShapeDtypeStruct((SP, HIDDEN), f32),
         jax.ShapeDtypeStruct((SP, DS), f32),
         jax.ShapeDtypeStruct((SP, DS), f32)) + tbl_out,
        (_rows((SB, HIDDEN)), _rows((SB, DS)), _rows((SB, DS))) + tbl_spec,
    )(hsum3, zcnt, degp, Wq[0], Wk[0], Wv[0], wet[0], Wc[0])

    ge = (EP // RB,)
    hc = hb
    for l in range(LAYERS):
        if l > 0:
            hc, zc, tdst, tsrca, tsrcc = _tc(
                _k_upd_body, gs,
                [_rows((SB, HIDDEN)), _rows((SB, DS)), _mid((NC, SB, 64)),
                 _mid((NC, SB, DS)), _rows((SB, DS)),
                 w_spec, w_spec, w_spec, w_spec, wet_spec, wc_spec],
                (jax.ShapeDtypeStruct((SP, HIDDEN), f32),
                 jax.ShapeDtypeStruct((SP, DS), f32)) + tbl_out,
                (_rows((SB, HIDDEN)), _rows((SB, DS))) + tbl_spec,
            )(hc, zc, agg3, dzp, degv, Wo[l - 1],
              Wq[l], Wk[l], Wv[l], wet[l], Wc[l])

        gdsta, gsrca = _sc_gather2(tdst, dst, tsrca, src, D_DST, D_SRC)
        logits = _tc(
            _k_logits_body, ge,
            [_rows((RB, D_DST)), _rows((RB, D_SRC)), _rows((RB, EDGE))],
            jax.ShapeDtypeStruct((EP, DS), f32), _rows((RB, DS)),
        )(gdsta, gsrca, ea_p)
        lsump = _sc_scatter_small(logits, dst, SP, z16)
        ctab = _tc(
            _k_ctab_body, gs, [_mid((NC, SB, DS)), _rows((SB, DS))],
            jax.ShapeDtypeStruct((SP, DS), f32), _rows((SB, DS)),
        )(lsump, degv)
        gct = _sc_gather1(ctab, dst, DS)
        w16 = _tc(
            _k_w_body, ge, [_rows((RB, DS)), _rows((RB, DS))],
            jax.ShapeDtypeStruct((EP, DS), f32), _rows((RB, DS)),
        )(logits, gct)
        denp = _sc_scatter_small(w16, dst, SP, z16)
        ddst = _tc(
            _k_ddst_body, gs, [_mid((NC, SB, DS)), _rows((SB, DS))],
            jax.ShapeDtypeStruct((SP, DS), f32), _rows((SB, DS)),
        )(denp, zc)
        gsrcc, gdd = _sc_gather2(tsrcc, src, ddst, dst, D_SRC, DS)
        msg3, dzv = _tc(
            _k_msg_body, ge,
            [_rows((RB, DS)), _rows((RB, DS)), _rows((RB, D_SRC))],
            (jax.ShapeDtypeStruct((NC, EP, 64), f32),
             jax.ShapeDtypeStruct((EP, DS), f32)),
            (_mid((NC, RB, 64)), _rows((RB, DS))),
        )(w16, gdd, gsrcc)
        agg3 = _sc_scatter_cols(msg3, dst, SP, z64)
        dzp = _sc_scatter_small(dzv, dst, SP, z16)

    br, br3 = _tc(
        _k_final_body, gs,
        [_rows((SB, HIDDEN)), _mid((NC, SB, 64)), w_spec],
        (jax.ShapeDtypeStruct((SP, HIDDEN), f32),
         jax.ShapeDtypeStruct((NC, SP, 64), f32)),
        (_rows((SB, HIDDEN)), _mid((NC, SB, 64))),
    )(hc, agg3, Wo[LAYERS - 1])

    grp = _sc_scatter_cols(br3, batch_p, SGP, z64g, ch=112)
    gr = _tc(
        _k_graph_body, (1,), [_full((NC, SGP, 64))],
        jax.ShapeDtypeStruct((SGP, HIDDEN), f32), _full((SGP, HIDDEN)),
    )(grp)

    return (hb[:N_BLOCK], br[:N_BLOCK], gr[:N_GRAPH])


# speedup vs baseline: 1.2773x; 1.1120x over previous
import functools

import jax
import jax.numpy as jnp
import numpy as np
from jax import lax
from jax.experimental import pallas as pl
from jax.experimental.pallas import tpu as pltpu
from jax.experimental.pallas import tpu_sc as plsc

N_UNIT = 100000
N_BLOCK = 25000
N_GRAPH = 64
E = 400000
HIDDEN = 128
NRBF = 16
EDGE = 16
NHEAD = 4
DH = HIDDEN // NHEAD
LAYERS = 3
CUTOFF = 7.0

NC, NS = 2, 16
NW = NC * NS
SP = 25088
EP = 409600
NUP = 102400
SGP = 128
DS = 16
D_DST = 272
D_SRC = 144

_mesh = lambda: plsc.VectorSubcoreMesh(core_axis_name="c", subcore_axis_name="s")


def _sc_gather2(tab1, idx1, tab2, idx2, d1, d2, ch=128):
    n = idx1.shape[0]
    per_w = n // NW
    iters = per_w // ch

    @functools.partial(
        pl.kernel, mesh=_mesh(),
        compiler_params=pltpu.CompilerParams(use_tc_tiling_on_sc=False),
        out_type=(jax.ShapeDtypeStruct((n, d1), jnp.float32),
                  jax.ShapeDtypeStruct((n, d2), jnp.float32)),
        scratch_types=[pltpu.VMEM((ch,), jnp.int32),
                       pltpu.VMEM((ch,), jnp.int32),
                       pltpu.VMEM((ch, d1), jnp.float32),
                       pltpu.VMEM((ch, d2), jnp.float32),
                       pltpu.SemaphoreType.DMA,
                       pltpu.SemaphoreType.DMA],
    )
    def k(t1, i1, t2, i2, o1, o2, iv1, iv2, r1, r2, s1, s2):
        wid = lax.axis_index("s") * NC + lax.axis_index("c")
        base = wid * per_w

        def body(i, carry):
            b = base + i * ch
            pltpu.sync_copy(i1.at[pl.ds(b, ch)], iv1)
            pltpu.sync_copy(i2.at[pl.ds(b, ch)], iv2)
            c1 = pltpu.async_copy(t1.at[iv1], r1, s1)
            c2 = pltpu.async_copy(t2.at[iv2], r2, s2)
            c1.wait()
            c2.wait()
            pltpu.sync_copy(r1, o1.at[pl.ds(b, ch)])
            pltpu.sync_copy(r2, o2.at[pl.ds(b, ch)])
            return carry

        lax.fori_loop(0, iters, body, 0)

    return k(tab1, idx1, tab2, idx2)


def _sc_gather1(tab, idx, d, ch=128):
    n = idx.shape[0]
    per_w = n // NW
    iters = per_w // ch

    @functools.partial(
        pl.kernel, mesh=_mesh(),
        compiler_params=pltpu.CompilerParams(use_tc_tiling_on_sc=False),
        out_type=jax.ShapeDtypeStruct((n, d), jnp.float32),
        scratch_types=[pltpu.VMEM((ch,), jnp.int32),
                       pltpu.VMEM((ch, d), jnp.float32),
                       pltpu.SemaphoreType.DMA],
    )
    def k(t1, i1, o1, iv1, r1, s1):
        wid = lax.axis_index("s") * NC + lax.axis_index("c")
        base = wid * per_w

        def body(i, carry):
            b = base + i * ch
            pltpu.sync_copy(i1.at[pl.ds(b, ch)], iv1)
            pltpu.async_copy(t1.at[iv1], r1, s1).wait()
            pltpu.sync_copy(r1, o1.at[pl.ds(b, ch)])
            return carry

        lax.fori_loop(0, iters, body, 0)

    return k(tab, idx)


def _sc_scatter_cols(vals3, idx, s_out, zeros64, ch=128):
    n = idx.shape[0]
    per_t = n // NS
    iters = per_t // ch
    stripe = s_out // NS

    @functools.partial(
        pl.kernel, mesh=_mesh(),
        compiler_params=pltpu.CompilerParams(use_tc_tiling_on_sc=False),
        out_type=jax.ShapeDtypeStruct((NC, s_out, 64), jnp.float32),
        scratch_types=[pltpu.VMEM((ch,), jnp.int32),
                       pltpu.VMEM((ch, 64), jnp.float32),
                       pltpu.VMEM_SHARED((s_out, 64), jnp.float32)],
    )
    def k(v_hbm, i_hbm, z_hbm, o_hbm, iv, vv, acc):
        cid = lax.axis_index("c")
        sid = lax.axis_index("s")
        r0 = sid * stripe
        pltpu.sync_copy(z_hbm.at[pl.ds(r0, stripe)], acc.at[pl.ds(r0, stripe)])
        plsc.subcore_barrier()

        def body(i, carry):
            b = sid * per_t + i * ch
            pltpu.sync_copy(i_hbm.at[pl.ds(b, ch)], iv)
            pltpu.sync_copy(v_hbm.at[cid, pl.ds(b, ch), :], vv)
            pltpu.sync_copy(vv, acc.at[iv], add=True)
            return carry

        lax.fori_loop(0, iters, body, 0)
        plsc.subcore_barrier()
        pltpu.sync_copy(acc.at[pl.ds(r0, stripe)],
                        o_hbm.at[cid, pl.ds(r0, stripe), :])

    return k(vals3, idx, zeros64)


def _sc_scatter_small(vals, idx, s_out, zeros16, ch=128):
    n = idx.shape[0]
    per_w = n // NW
    iters = per_w // ch
    stripe = s_out // NS

    @functools.partial(
        pl.kernel, mesh=_mesh(),
        compiler_params=pltpu.CompilerParams(use_tc_tiling_on_sc=False),
        out_type=jax.ShapeDtypeStruct((NC, s_out, DS), jnp.float32),
        scratch_types=[pltpu.VMEM((ch,), jnp.int32),
                       pltpu.VMEM((ch, DS), jnp.float32),
                       pltpu.VMEM_SHARED((s_out, DS), jnp.float32)],
    )
    def k(v_hbm, i_hbm, z_hbm, o_hbm, iv, vv, acc):
        cid = lax.axis_index("c")
        sid = lax.axis_index("s")
        wid = sid * NC + cid
        r0 = sid * stripe
        pltpu.sync_copy(z_hbm.at[pl.ds(r0, stripe)], acc.at[pl.ds(r0, stripe)])
        plsc.subcore_barrier()

        def body(i, carry):
            b = wid * per_w + i * ch
            pltpu.sync_copy(i_hbm.at[pl.ds(b, ch)], iv)
            pltpu.sync_copy(v_hbm.at[pl.ds(b, ch), :], vv)
            pltpu.sync_copy(vv, acc.at[iv], add=True)
            return carry

        lax.fori_loop(0, iters, body, 0)
        plsc.subcore_barrier()
        pltpu.sync_copy(acc.at[pl.ds(r0, stripe)],
                        o_hbm.at[cid, pl.ds(r0, stripe), :])

    return k(vals, idx, zeros16)


def _tc(body, grid, in_specs, out_shapes, out_specs):
    return pl.pallas_call(
        body, grid=grid, in_specs=in_specs,
        out_shape=out_shapes, out_specs=out_specs)


def _full(shape):
    return pl.BlockSpec(shape, lambda i: tuple(0 for _ in shape))


def _rows(shape):
    nd = len(shape)
    return pl.BlockSpec(shape, lambda i: (i,) + (0,) * (nd - 1))


def _mid(shape):
    return pl.BlockSpec(shape, lambda i: (0, i, 0))


SB = 784
RB = 2048


def _tables(hb, zc, wq, wk, wv, wet, wc, tdst_ref, tsrca_ref, tsrcc_ref):
    q = jnp.dot(hb, wq, preferred_element_type=jnp.float32)
    kk = jnp.dot(hb, wk, preferred_element_type=jnp.float32)
    v = jnp.dot(hb, wv, preferred_element_type=jnp.float32)
    a = jnp.concatenate(
        [jnp.dot(q[:, 32 * h:32 * h + 32], wet[32 * h:32 * h + 32, :],
                 preferred_element_type=jnp.float32) for h in range(NHEAD)],
        axis=1)
    vc = jnp.sum((v * wc[:, 0][None, :]).reshape(-1, NHEAD, DH), axis=-1)
    z3 = zc[:, 0:3]
    zpad = jnp.zeros((hb.shape[0], D_DST - 259), jnp.float32)
    tdst_ref[...] = jnp.concatenate([q, a, z3, zpad], axis=1)
    spad = jnp.zeros((hb.shape[0], D_SRC - 131), jnp.float32)
    tsrca_ref[...] = jnp.concatenate([kk, z3, spad], axis=1)
    cpad = jnp.zeros((hb.shape[0], D_SRC - 135), jnp.float32)
    tsrcc_ref[...] = jnp.concatenate([v, vc, z3, cpad], axis=1)


def _k_pool0_body(hs_ref, zcnt_ref, degp_ref, wq_ref, wk_ref, wv_ref,
                  wet_ref, wc_ref, hb_ref, zc_ref, degv_ref,
                  tdst_ref, tsrca_ref, tsrcc_ref):
    zc01 = zcnt_ref[0] + zcnt_ref[1]
    cnt = jnp.maximum(zc01[:, 3:4], 1.0)
    hb = jnp.concatenate([hs_ref[0], hs_ref[1]], axis=1) / cnt
    z3 = zc01[:, 0:3] / cnt
    deg01 = degp_ref[0] + degp_ref[1]
    deg = jnp.maximum(deg01[:, 0:1], 1.0)
    pad13 = jnp.zeros((SB, DS - 3), jnp.float32)
    zc = jnp.concatenate([z3, pad13], axis=1)
    hb_ref[...] = hb
    zc_ref[...] = zc
    pad15 = jnp.zeros((SB, DS - 1), jnp.float32)
    degv_ref[...] = jnp.concatenate([deg, pad15], axis=1)
    _tables(hb, zc, wq_ref[...], wk_ref[...], wv_ref[...], wet_ref[...],
            wc_ref[...], tdst_ref, tsrca_ref, tsrcc_ref)


def _k_upd_body(hc_ref, zc_ref, agg_ref, dzp_ref, degv_ref, wo_ref,
                wq_ref, wk_ref, wv_ref, wet_ref, wc_ref,
                hcn_ref, zcn_ref, tdst_ref, tsrca_ref, tsrcc_ref):
    agg = jnp.concatenate([agg_ref[0], agg_ref[1]], axis=1)
    hc = hc_ref[...] + jnp.dot(agg, wo_ref[...],
                               preferred_element_type=jnp.float32)
    dz01 = dzp_ref[0] + dzp_ref[1]
    deg = degv_ref[...][:, 0:1]
    z3 = zc_ref[...][:, 0:3] + dz01[:, 0:3] / deg
    pad13 = jnp.zeros((SB, DS - 3), jnp.float32)
    zc = jnp.concatenate([z3, pad13], axis=1)
    hcn_ref[...] = hc
    zcn_ref[...] = zc
    _tables(hc, zc, wq_ref[...], wk_ref[...], wv_ref[...], wet_ref[...],
            wc_ref[...], tdst_ref, tsrca_ref, tsrcc_ref)


_RBF_DEN = 2.0 * (CUTOFF / NRBF) ** 2
_ISQ = 1.0 / np.sqrt(DH)


def _k_logits_body(gd_ref, gs_ref, ea_ref, lo_ref):
    i = pl.program_id(0)
    gd = gd_ref[...]
    gs = gs_ref[...]
    rel = gd[:, 256:259] - gs[:, 128:131]
    d = jnp.sqrt(jnp.sum(rel * rel, axis=1, keepdims=True) + 1e-8)
    centers = (lax.broadcasted_iota(jnp.int32, (1, NRBF), 1).astype(jnp.float32)
               * (CUTOFF / (NRBF - 1)))
    rbf = jnp.exp(-((d - centers) ** 2) / _RBF_DEN)
    feat = jnp.concatenate([rbf, ea_ref[...]], axis=1)
    qk = jnp.sum((gd[:, 0:128] * gs[:, 0:128]).reshape(RB, NHEAD, DH), axis=-1)
    fa = jnp.sum(gd[:, 128:256].reshape(RB, NHEAD, DH) * feat[:, None, :],
                 axis=-1)
    w = jnp.exp((qk + fa) * _ISQ)
    rows = i * RB + lax.broadcasted_iota(jnp.int32, (RB, 1), 0)
    w = jnp.where(rows < E, w, 0.0)
    lo_ref[...] = jnp.concatenate(
        [w, jnp.zeros((RB, DS - NHEAD), jnp.float32)], axis=1)


def _k_ctab_body(lsum_ref, degv_ref, ct_ref):
    lsum = lsum_ref[0] + lsum_ref[1]
    c = lsum[:, 0:NHEAD] / degv_ref[...][:, 0:1]
    ct_ref[...] = jnp.concatenate(
        [c, jnp.zeros((SB, DS - NHEAD), jnp.float32)], axis=1)


def _k_w_body(lo_ref, gc_ref, w_ref):
    i = pl.program_id(0)
    w = jnp.exp(lo_ref[...][:, 0:NHEAD] - gc_ref[...][:, 0:NHEAD])
    rows = i * RB + lax.broadcasted_iota(jnp.int32, (RB, 1), 0)
    w = jnp.where(rows < E, w, 0.0)
    w_ref[...] = jnp.concatenate(
        [w, jnp.zeros((RB, DS - NHEAD), jnp.float32)], axis=1)


def _k_ddst_body(denp_ref, zc_ref, dd_ref):
    den = denp_ref[0] + denp_ref[1]
    dd = den[:, 0:NHEAD] + 1e-8
    z3 = zc_ref[...][:, 0:3]
    dd_ref[...] = jnp.concatenate(
        [dd, z3, jnp.zeros((SB, DS - NHEAD - 3), jnp.float32)], axis=1)


def _k_msg_body(w_ref, gdd_ref, gs_ref, msg_ref, dzv_ref):
    gdd = gdd_ref[...]
    gs = gs_ref[...]
    alpha = w_ref[...][:, 0:NHEAD] / gdd[:, 0:NHEAD]
    v = gs[:, 0:128].reshape(RB, NHEAD, DH)
    msg = (alpha[:, :, None] * v).reshape(RB, HIDDEN)
    msg_ref[...] = jnp.stack([msg[:, 0:64], msg[:, 64:128]], axis=0)
    csc = jnp.sum(alpha * gs[:, 128:132], axis=1, keepdims=True)
    rel = gdd[:, NHEAD:NHEAD + 3] - gs[:, 132:135]
    dzv_ref[...] = jnp.concatenate(
        [rel * csc, jnp.zeros((RB, DS - 3), jnp.float32)], axis=1)


def _k_final_body(hc_ref, agg_ref, wo_ref, br_ref, br3_ref):
    agg = jnp.concatenate([agg_ref[0], agg_ref[1]], axis=1)
    hc = hc_ref[...] + jnp.dot(agg, wo_ref[...],
                               preferred_element_type=jnp.float32)
    nrm = jnp.sqrt(jnp.sum(hc * hc, axis=1, keepdims=True))
    br = hc / (nrm + 1e-12)
    br_ref[...] = br
    br3_ref[...] = jnp.stack([br[:, 0:64], br[:, 64:128]], axis=0)


def _k_graph_body(grp_ref, gr_ref):
    g = jnp.concatenate([grp_ref[0], grp_ref[1]], axis=1)
    nrm = jnp.sqrt(jnp.sum(g * g, axis=1, keepdims=True))
    gr_ref[...] = g / (nrm + 1e-12)


def kernel(H, Z, block_id, batch_id, edges, edge_attr, Wq, Wk, Wv, We, Wo, Wc):
    f32 = jnp.float32
    src = jnp.pad(edges[0].astype(jnp.int32), (0, EP - E))
    dst = jnp.pad(edges[1].astype(jnp.int32), (0, EP - E))
    ea_p = jnp.pad(edge_attr, ((0, EP - E), (0, 0)))
    emask = (jnp.arange(EP) < E).astype(f32)[:, None]
    degvals = jnp.pad(emask, ((0, 0), (0, DS - 1)))
    bid_p = jnp.pad(block_id.astype(jnp.int32), (0, NUP - N_UNIT))
    hp = jnp.pad(H, ((0, NUP - N_UNIT), (0, 0)))
    h3 = jnp.stack([hp[:, 0:64], hp[:, 64:128]], axis=0)
    zuv = jnp.pad(
        jnp.concatenate([Z.reshape(N_UNIT, 3), jnp.ones((N_UNIT, 1), f32)],
                        axis=1),
        ((0, NUP - N_UNIT), (0, DS - 4)))
    batch_p = jnp.pad(batch_id.astype(jnp.int32), (0, SP - N_BLOCK))
    z64 = jnp.zeros((SP, 64), f32)
    z64g = jnp.zeros((SGP, 64), f32)
    z16 = jnp.zeros((SP, DS), f32)
    wet = jnp.transpose(We, (0, 2, 1))

    hsum3 = _sc_scatter_cols(h3, bid_p, SP, z64)
    zcnt = _sc_scatter_small(zuv, bid_p, SP, z16)
    degp = _sc_scatter_small(degvals, dst, SP, z16)

    gs = (32,)
    w_spec = _full((HIDDEN, HIDDEN))
    wet_spec = _full((HIDDEN, 32))
    wc_spec = _full((HIDDEN, 1))
    tbl_out = (jax.ShapeDtypeStruct((SP, D_DST), f32),
               jax.ShapeDtypeStruct((SP, D_SRC), f32),
               jax.ShapeDtypeStruct((SP, D_SRC), f32))
    tbl_spec = (_rows((SB, D_DST)), _rows((SB, D_SRC)), _rows((SB, D_SRC)))

    hb, zc, degv, tdst, tsrca, tsrcc = _tc(
        _k_pool0_body, gs,
        [_mid((NC, SB, 64)), _mid((NC, SB, DS)), _mid((NC, SB, DS)),
         w_spec, w_spec, w_spec, wet_spec, wc_spec],
        (jax.ShapeDtypeStruct((SP, HIDDEN), f32),
         jax.ShapeDtypeStruct((SP, DS), f32),
         jax.ShapeDtypeStruct((SP, DS), f32)) + tbl_out,
        (_rows((SB, HIDDEN)), _rows((SB, DS)), _rows((SB, DS))) + tbl_spec,
    )(hsum3, zcnt, degp, Wq[0], Wk[0], Wv[0], wet[0], Wc[0])

    ge = (EP // RB,)
    hc = hb
    for l in range(LAYERS):
        if l > 0:
            hc, zc, tdst, tsrca, tsrcc = _tc(
                _k_upd_body, gs,
                [_rows((SB, HIDDEN)), _rows((SB, DS)), _mid((NC, SB, 64)),
                 _mid((NC, SB, DS)), _rows((SB, DS)),
                 w_spec, w_spec, w_spec, w_spec, wet_spec, wc_spec],
                (jax.ShapeDtypeStruct((SP, HIDDEN), f32),
                 jax.ShapeDtypeStruct((SP, DS), f32)) + tbl_out,
                (_rows((SB, HIDDEN)), _rows((SB, DS))) + tbl_spec,
            )(hc, zc, agg3, dzp, degv, Wo[l - 1],
              Wq[l], Wk[l], Wv[l], wet[l], Wc[l])

        gdsta, gsrca = _sc_gather2(tdst, dst, tsrca, src, D_DST, D_SRC)
        w16 = _tc(
            _k_logits_body, ge,
            [_rows((RB, D_DST)), _rows((RB, D_SRC)), _rows((RB, EDGE))],
            jax.ShapeDtypeStruct((EP, DS), f32), _rows((RB, DS)),
        )(gdsta, gsrca, ea_p)
        denp = _sc_scatter_small(w16, dst, SP, z16)
        ddst = _tc(
            _k_ddst_body, gs, [_mid((NC, SB, DS)), _rows((SB, DS))],
            jax.ShapeDtypeStruct((SP, DS), f32), _rows((SB, DS)),
        )(denp, zc)
        gsrcc, gdd = _sc_gather2(tsrcc, src, ddst, dst, D_SRC, DS)
        msg3, dzv = _tc(
            _k_msg_body, ge,
            [_rows((RB, DS)), _rows((RB, DS)), _rows((RB, D_SRC))],
            (jax.ShapeDtypeStruct((NC, EP, 64), f32),
             jax.ShapeDtypeStruct((EP, DS), f32)),
            (_mid((NC, RB, 64)), _rows((RB, DS))),
        )(w16, gdd, gsrcc)
        agg3 = _sc_scatter_cols(msg3, dst, SP, z64)
        dzp = _sc_scatter_small(dzv, dst, SP, z16)

    br, br3 = _tc(
        _k_final_body, gs,
        [_rows((SB, HIDDEN)), _mid((NC, SB, 64)), w_spec],
        (jax.ShapeDtypeStruct((SP, HIDDEN), f32),
         jax.ShapeDtypeStruct((NC, SP, 64), f32)),
        (_rows((SB, HIDDEN)), _mid((NC, SB, 64))),
    )(hc, agg3, Wo[LAYERS - 1])

    grp = _sc_scatter_cols(br3, batch_p, SGP, z64g, ch=112)
    gr = _tc(
        _k_graph_body, (1,), [_full((NC, SGP, 64))],
        jax.ShapeDtypeStruct((SGP, HIDDEN), f32), _full((SGP, HIDDEN)),
    )(grp)

    return (hb[:N_BLOCK], br[:N_BLOCK], gr[:N_GRAPH])


# speedup vs baseline: 1.7586x; 1.3768x over previous
import functools

import jax
import jax.numpy as jnp
import numpy as np
from jax import lax
from jax.experimental import pallas as pl
from jax.experimental.pallas import tpu as pltpu
from jax.experimental.pallas import tpu_sc as plsc

N_UNIT = 100000
N_BLOCK = 25000
N_GRAPH = 64
E = 400000
HIDDEN = 128
NRBF = 16
EDGE = 16
NHEAD = 4
DH = HIDDEN // NHEAD
LAYERS = 3
CUTOFF = 7.0

NC, NS = 2, 16
NW = NC * NS
SP = 25088
EP = 409600
NUP = 102400
SGP = 128
DS = 16
D_DST = 272
D_SRC = 144

_mesh = lambda: plsc.VectorSubcoreMesh(core_axis_name="c", subcore_axis_name="s")


def _sc_gather2(tab1, idx1, tab2, idx2, d1, d2, ch=128):
    n = idx1.shape[0]
    per_w = n // NW
    iters = per_w // ch

    @functools.partial(
        pl.kernel, mesh=_mesh(),
        compiler_params=pltpu.CompilerParams(
            use_tc_tiling_on_sc=False, needs_layout_passes=False),
        out_type=(jax.ShapeDtypeStruct((n, d1), jnp.float32),
                  jax.ShapeDtypeStruct((n, d2), jnp.float32)),
        scratch_types=[pltpu.VMEM((ch,), jnp.int32),
                       pltpu.VMEM((ch,), jnp.int32),
                       pltpu.VMEM((ch, d1), jnp.float32),
                       pltpu.VMEM((ch, d2), jnp.float32),
                       pltpu.SemaphoreType.DMA,
                       pltpu.SemaphoreType.DMA],
    )
    def k(t1, i1, t2, i2, o1, o2, iv1, iv2, r1, r2, s1, s2):
        wid = lax.axis_index("s") * NC + lax.axis_index("c")
        base = wid * per_w

        def body(i, carry):
            b = base + i * ch
            pltpu.sync_copy(i1.at[pl.ds(b, ch)], iv1)
            pltpu.sync_copy(i2.at[pl.ds(b, ch)], iv2)
            c1 = pltpu.async_copy(t1.at[iv1], r1, s1)
            c2 = pltpu.async_copy(t2.at[iv2], r2, s2)
            c1.wait()
            c2.wait()
            pltpu.sync_copy(r1, o1.at[pl.ds(b, ch)])
            pltpu.sync_copy(r2, o2.at[pl.ds(b, ch)])
            return carry

        lax.fori_loop(0, iters, body, 0)

    return k(tab1, idx1, tab2, idx2)


def _sc_gather1(tab, idx, d, ch=128):
    n = idx.shape[0]
    per_w = n // NW
    iters = per_w // ch

    @functools.partial(
        pl.kernel, mesh=_mesh(),
        compiler_params=pltpu.CompilerParams(
            use_tc_tiling_on_sc=False, needs_layout_passes=False),
        out_type=jax.ShapeDtypeStruct((n, d), jnp.float32),
        scratch_types=[pltpu.VMEM((ch,), jnp.int32),
                       pltpu.VMEM((ch, d), jnp.float32),
                       pltpu.SemaphoreType.DMA],
    )
    def k(t1, i1, o1, iv1, r1, s1):
        wid = lax.axis_index("s") * NC + lax.axis_index("c")
        base = wid * per_w

        def body(i, carry):
            b = base + i * ch
            pltpu.sync_copy(i1.at[pl.ds(b, ch)], iv1)
            pltpu.async_copy(t1.at[iv1], r1, s1).wait()
            pltpu.sync_copy(r1, o1.at[pl.ds(b, ch)])
            return carry

        lax.fori_loop(0, iters, body, 0)

    return k(tab, idx)


def _sc_scatter_cols(vals3, idx, s_out, zeros64, ch=128):
    n = idx.shape[0]
    per_t = n // NS
    iters = per_t // ch
    stripe = s_out // NS

    @functools.partial(
        pl.kernel, mesh=_mesh(),
        compiler_params=pltpu.CompilerParams(
            use_tc_tiling_on_sc=False, needs_layout_passes=False),
        out_type=jax.ShapeDtypeStruct((NC, s_out, 64), jnp.float32),
        scratch_types=[pltpu.VMEM((ch,), jnp.int32),
                       pltpu.VMEM((ch, 64), jnp.float32),
                       pltpu.VMEM_SHARED((s_out, 64), jnp.float32)],
    )
    def k(v_hbm, i_hbm, z_hbm, o_hbm, iv, vv, acc):
        cid = lax.axis_index("c")
        sid = lax.axis_index("s")
        r0 = sid * stripe
        pltpu.sync_copy(z_hbm.at[pl.ds(r0, stripe)], acc.at[pl.ds(r0, stripe)])
        plsc.subcore_barrier()

        def body(i, carry):
            b = sid * per_t + i * ch
            pltpu.sync_copy(i_hbm.at[pl.ds(b, ch)], iv)
            pltpu.sync_copy(v_hbm.at[cid, pl.ds(b, ch), :], vv)
            pltpu.sync_copy(vv, acc.at[iv], add=True)
            return carry

        lax.fori_loop(0, iters, body, 0)
        plsc.subcore_barrier()
        pltpu.sync_copy(acc.at[pl.ds(r0, stripe)],
                        o_hbm.at[cid, pl.ds(r0, stripe), :])

    return k(vals3, idx, zeros64)


def _sc_scatter_small(vals, idx, s_out, zeros16, ch=128):
    n = idx.shape[0]
    per_w = n // NW
    iters = per_w // ch
    stripe = s_out // NS

    @functools.partial(
        pl.kernel, mesh=_mesh(),
        compiler_params=pltpu.CompilerParams(
            use_tc_tiling_on_sc=False, needs_layout_passes=False),
        out_type=jax.ShapeDtypeStruct((NC, s_out, DS), jnp.float32),
        scratch_types=[pltpu.VMEM((ch,), jnp.int32),
                       pltpu.VMEM((ch, DS), jnp.float32),
                       pltpu.VMEM_SHARED((s_out, DS), jnp.float32)],
    )
    def k(v_hbm, i_hbm, z_hbm, o_hbm, iv, vv, acc):
        cid = lax.axis_index("c")
        sid = lax.axis_index("s")
        wid = sid * NC + cid
        r0 = sid * stripe
        pltpu.sync_copy(z_hbm.at[pl.ds(r0, stripe)], acc.at[pl.ds(r0, stripe)])
        plsc.subcore_barrier()

        def body(i, carry):
            b = wid * per_w + i * ch
            pltpu.sync_copy(i_hbm.at[pl.ds(b, ch)], iv)
            pltpu.sync_copy(v_hbm.at[pl.ds(b, ch), :], vv)
            pltpu.sync_copy(vv, acc.at[iv], add=True)
            return carry

        lax.fori_loop(0, iters, body, 0)
        plsc.subcore_barrier()
        pltpu.sync_copy(acc.at[pl.ds(r0, stripe)],
                        o_hbm.at[cid, pl.ds(r0, stripe), :])

    return k(vals, idx, zeros16)


def _vsqrt(r2):
    ib = plsc.bitcast(r2, jnp.int32)
    x = plsc.bitcast((ib >> 1) + jnp.int32(0x1FBD1DF5), jnp.float32)
    for _ in range(3):
        x = 0.5 * (x + r2 / x)
    return x


def _sc_pass_a(tdst, tsrca, dst, src, ea, z16, ch=128):
    per_w = EP // NW
    iters = per_w // ch
    stripe = SP // NS
    isq = float(1.0 / np.sqrt(DH))
    inv2s = float(1.0 / _RBF_DEN)
    cstep = float(CUTOFF / (NRBF - 1))

    @functools.partial(
        pl.kernel, mesh=_mesh(),
        compiler_params=pltpu.CompilerParams(
            use_tc_tiling_on_sc=False, needs_layout_passes=False),
        out_type=(jax.ShapeDtypeStruct((EP, DS), jnp.float32),
                  jax.ShapeDtypeStruct((NC, SP, DS), jnp.float32)),
        scratch_types=[pltpu.VMEM((ch,), jnp.int32),
                       pltpu.VMEM((ch,), jnp.int32),
                       pltpu.VMEM((ch, D_DST), jnp.float32),
                       pltpu.VMEM((ch, D_SRC), jnp.float32),
                       pltpu.VMEM((ch, EDGE), jnp.float32),
                       pltpu.VMEM((ch, DS), jnp.float32),
                       pltpu.VMEM_SHARED((SP, DS), jnp.float32),
                       pltpu.SemaphoreType.DMA,
                       pltpu.SemaphoreType.DMA],
    )
    def k(td_h, ts_h, di_h, si_h, ea_h, z_h, w_h, dp_h,
          dvi, svi, gd, gs, eab, wb, dacc, s1, s2):
        cid = lax.axis_index("c")
        sid = lax.axis_index("s")
        wid = sid * NC + cid
        r0 = sid * stripe
        pltpu.sync_copy(z_h.at[pl.ds(r0, stripe)], dacc.at[pl.ds(r0, stripe)])
        pltpu.sync_copy(z_h.at[pl.ds(0, ch)], wb)
        plsc.subcore_barrier()
        lanes = lax.iota(jnp.int32, 16)

        def chunk(i, carry):
            b = wid * per_w + i * ch
            pltpu.sync_copy(di_h.at[pl.ds(b, ch)], dvi)
            pltpu.sync_copy(si_h.at[pl.ds(b, ch)], svi)
            c1 = pltpu.async_copy(td_h.at[dvi], gd, s1)
            c2 = pltpu.async_copy(ts_h.at[svi], gs, s2)
            c1.wait()
            c2.wait()
            pltpu.sync_copy(ea_h.at[pl.ds(b, ch), :], eab)

            def group(g, c3):
                rows = g * 16 + lanes

                def col(ref, c):
                    return plsc.load_gather(
                        ref, [rows, jnp.full((16,), c, jnp.int32)])

                rel2 = jnp.full((16,), 1e-8, jnp.float32)
                for t in range(3):
                    dt = col(gd, 256 + t) - col(gs, 128 + t)
                    rel2 = rel2 + dt * dt
                d = _vsqrt(rel2)
                feat = []
                for j in range(NRBF):
                    t = d - (cstep * j)
                    feat.append(jnp.exp(-(t * t) * inv2s))
                for j in range(EDGE):
                    feat.append(col(eab, j))
                valid = (b + rows) < E
                for h in range(NHEAD):
                    acc = jnp.zeros((16,), jnp.float32)
                    for c in range(DH):
                        acc = acc + col(gd, 32 * h + c) * col(gs, 32 * h + c)
                    for j in range(32):
                        acc = acc + feat[j] * col(gd, 128 + 32 * h + j)
                    w = jnp.exp(acc * isq)
                    w = jnp.where(valid, w, 0.0)
                    plsc.store_scatter(
                        wb, [rows, jnp.full((16,), h, jnp.int32)], w)
                return c3

            lax.fori_loop(0, ch // 16, group, 0)
            pltpu.sync_copy(wb, w_h.at[pl.ds(b, ch)])
            pltpu.sync_copy(wb, dacc.at[dvi], add=True)
            return carry

        lax.fori_loop(0, iters, chunk, 0)
        plsc.subcore_barrier()
        pltpu.sync_copy(dacc.at[pl.ds(r0, stripe)],
                        dp_h.at[cid, pl.ds(r0, stripe), :])

    return k(tdst, tsrca, dst, src, ea, z16)


def _tc(body, grid, in_specs, out_shapes, out_specs):
    return pl.pallas_call(
        body, grid=grid, in_specs=in_specs,
        out_shape=out_shapes, out_specs=out_specs)


def _full(shape):
    return pl.BlockSpec(shape, lambda i: tuple(0 for _ in shape))


def _rows(shape):
    nd = len(shape)
    return pl.BlockSpec(shape, lambda i: (i,) + (0,) * (nd - 1))


def _mid(shape):
    return pl.BlockSpec(shape, lambda i: (0, i, 0))


SB = 784
RB = 2048


def _tables(hb, zc, wq, wk, wv, wet, wc, tdst_ref, tsrca_ref, tsrcc_ref):
    q = jnp.dot(hb, wq, preferred_element_type=jnp.float32)
    kk = jnp.dot(hb, wk, preferred_element_type=jnp.float32)
    v = jnp.dot(hb, wv, preferred_element_type=jnp.float32)
    a = jnp.concatenate(
        [jnp.dot(q[:, 32 * h:32 * h + 32], wet[32 * h:32 * h + 32, :],
                 preferred_element_type=jnp.float32) for h in range(NHEAD)],
        axis=1)
    vc = jnp.sum((v * wc[:, 0][None, :]).reshape(-1, NHEAD, DH), axis=-1)
    z3 = zc[:, 0:3]
    zpad = jnp.zeros((hb.shape[0], D_DST - 259), jnp.float32)
    tdst_ref[...] = jnp.concatenate([q, a, z3, zpad], axis=1)
    spad = jnp.zeros((hb.shape[0], D_SRC - 131), jnp.float32)
    tsrca_ref[...] = jnp.concatenate([kk, z3, spad], axis=1)
    cpad = jnp.zeros((hb.shape[0], D_SRC - 135), jnp.float32)
    tsrcc_ref[...] = jnp.concatenate([v, vc, z3, cpad], axis=1)


def _k_pool0_body(hs_ref, zcnt_ref, degp_ref, wq_ref, wk_ref, wv_ref,
                  wet_ref, wc_ref, hb_ref, zc_ref, degv_ref,
                  tdst_ref, tsrca_ref, tsrcc_ref):
    zc01 = zcnt_ref[0] + zcnt_ref[1]
    cnt = jnp.maximum(zc01[:, 3:4], 1.0)
    hb = jnp.concatenate([hs_ref[0], hs_ref[1]], axis=1) / cnt
    z3 = zc01[:, 0:3] / cnt
    deg01 = degp_ref[0] + degp_ref[1]
    deg = jnp.maximum(deg01[:, 0:1], 1.0)
    pad13 = jnp.zeros((SB, DS - 3), jnp.float32)
    zc = jnp.concatenate([z3, pad13], axis=1)
    hb_ref[...] = hb
    zc_ref[...] = zc
    pad15 = jnp.zeros((SB, DS - 1), jnp.float32)
    degv_ref[...] = jnp.concatenate([deg, pad15], axis=1)
    _tables(hb, zc, wq_ref[...], wk_ref[...], wv_ref[...], wet_ref[...],
            wc_ref[...], tdst_ref, tsrca_ref, tsrcc_ref)


def _k_upd_body(hc_ref, zc_ref, agg_ref, dzp_ref, degv_ref, wo_ref,
                wq_ref, wk_ref, wv_ref, wet_ref, wc_ref,
                hcn_ref, zcn_ref, tdst_ref, tsrca_ref, tsrcc_ref):
    agg = jnp.concatenate([agg_ref[0], agg_ref[1]], axis=1)
    hc = hc_ref[...] + jnp.dot(agg, wo_ref[...],
                               preferred_element_type=jnp.float32)
    dz01 = dzp_ref[0] + dzp_ref[1]
    deg = degv_ref[...][:, 0:1]
    z3 = zc_ref[...][:, 0:3] + dz01[:, 0:3] / deg
    pad13 = jnp.zeros((SB, DS - 3), jnp.float32)
    zc = jnp.concatenate([z3, pad13], axis=1)
    hcn_ref[...] = hc
    zcn_ref[...] = zc
    _tables(hc, zc, wq_ref[...], wk_ref[...], wv_ref[...], wet_ref[...],
            wc_ref[...], tdst_ref, tsrca_ref, tsrcc_ref)


_RBF_DEN = 2.0 * (CUTOFF / NRBF) ** 2
_ISQ = 1.0 / np.sqrt(DH)


def _k_logits_body(gd_ref, gs_ref, ea_ref, lo_ref):
    i = pl.program_id(0)
    gd = gd_ref[...]
    gs = gs_ref[...]
    rel = gd[:, 256:259] - gs[:, 128:131]
    d = jnp.sqrt(jnp.sum(rel * rel, axis=1, keepdims=True) + 1e-8)
    centers = (lax.broadcasted_iota(jnp.int32, (1, NRBF), 1).astype(jnp.float32)
               * (CUTOFF / (NRBF - 1)))
    rbf = jnp.exp(-((d - centers) ** 2) / _RBF_DEN)
    feat = jnp.concatenate([rbf, ea_ref[...]], axis=1)
    qk = jnp.sum((gd[:, 0:128] * gs[:, 0:128]).reshape(RB, NHEAD, DH), axis=-1)
    fa = jnp.sum(gd[:, 128:256].reshape(RB, NHEAD, DH) * feat[:, None, :],
                 axis=-1)
    w = jnp.exp((qk + fa) * _ISQ)
    rows = i * RB + lax.broadcasted_iota(jnp.int32, (RB, 1), 0)
    w = jnp.where(rows < E, w, 0.0)
    lo_ref[...] = jnp.concatenate(
        [w, jnp.zeros((RB, DS - NHEAD), jnp.float32)], axis=1)


def _k_ctab_body(lsum_ref, degv_ref, ct_ref):
    lsum = lsum_ref[0] + lsum_ref[1]
    c = lsum[:, 0:NHEAD] / degv_ref[...][:, 0:1]
    ct_ref[...] = jnp.concatenate(
        [c, jnp.zeros((SB, DS - NHEAD), jnp.float32)], axis=1)


def _k_w_body(lo_ref, gc_ref, w_ref):
    i = pl.program_id(0)
    w = jnp.exp(lo_ref[...][:, 0:NHEAD] - gc_ref[...][:, 0:NHEAD])
    rows = i * RB + lax.broadcasted_iota(jnp.int32, (RB, 1), 0)
    w = jnp.where(rows < E, w, 0.0)
    w_ref[...] = jnp.concatenate(
        [w, jnp.zeros((RB, DS - NHEAD), jnp.float32)], axis=1)


def _k_ddst_body(denp_ref, zc_ref, dd_ref):
    den = denp_ref[0] + denp_ref[1]
    dd = den[:, 0:NHEAD] + 1e-8
    z3 = zc_ref[...][:, 0:3]
    dd_ref[...] = jnp.concatenate(
        [dd, z3, jnp.zeros((SB, DS - NHEAD - 3), jnp.float32)], axis=1)


def _k_msg_body(w_ref, gdd_ref, gs_ref, msg_ref, dzv_ref):
    gdd = gdd_ref[...]
    gs = gs_ref[...]
    alpha = w_ref[...][:, 0:NHEAD] / gdd[:, 0:NHEAD]
    v = gs[:, 0:128].reshape(RB, NHEAD, DH)
    msg = (alpha[:, :, None] * v).reshape(RB, HIDDEN)
    msg_ref[...] = jnp.stack([msg[:, 0:64], msg[:, 64:128]], axis=0)
    csc = jnp.sum(alpha * gs[:, 128:132], axis=1, keepdims=True)
    rel = gdd[:, NHEAD:NHEAD + 3] - gs[:, 132:135]
    dzv_ref[...] = jnp.concatenate(
        [rel * csc, jnp.zeros((RB, DS - 3), jnp.float32)], axis=1)


def _k_final_body(hc_ref, agg_ref, wo_ref, br_ref, br3_ref):
    agg = jnp.concatenate([agg_ref[0], agg_ref[1]], axis=1)
    hc = hc_ref[...] + jnp.dot(agg, wo_ref[...],
                               preferred_element_type=jnp.float32)
    nrm = jnp.sqrt(jnp.sum(hc * hc, axis=1, keepdims=True))
    br = hc / (nrm + 1e-12)
    br_ref[...] = br
    br3_ref[...] = jnp.stack([br[:, 0:64], br[:, 64:128]], axis=0)


def _k_graph_body(grp_ref, gr_ref):
    g = jnp.concatenate([grp_ref[0], grp_ref[1]], axis=1)
    nrm = jnp.sqrt(jnp.sum(g * g, axis=1, keepdims=True))
    gr_ref[...] = g / (nrm + 1e-12)


def kernel(H, Z, block_id, batch_id, edges, edge_attr, Wq, Wk, Wv, We, Wo, Wc):
    f32 = jnp.float32
    src = jnp.pad(edges[0].astype(jnp.int32), (0, EP - E))
    dst = jnp.pad(edges[1].astype(jnp.int32), (0, EP - E))
    ea_p = jnp.pad(edge_attr, ((0, EP - E), (0, 0)))
    emask = (jnp.arange(EP) < E).astype(f32)[:, None]
    degvals = jnp.pad(emask, ((0, 0), (0, DS - 1)))
    bid_p = jnp.pad(block_id.astype(jnp.int32), (0, NUP - N_UNIT))
    hp = jnp.pad(H, ((0, NUP - N_UNIT), (0, 0)))
    h3 = jnp.stack([hp[:, 0:64], hp[:, 64:128]], axis=0)
    zuv = jnp.pad(
        jnp.concatenate([Z.reshape(N_UNIT, 3), jnp.ones((N_UNIT, 1), f32)],
                        axis=1),
        ((0, NUP - N_UNIT), (0, DS - 4)))
    batch_p = jnp.pad(batch_id.astype(jnp.int32), (0, SP - N_BLOCK))
    z64 = jnp.zeros((SP, 64), f32)
    z64g = jnp.zeros((SGP, 64), f32)
    z16 = jnp.zeros((SP, DS), f32)
    wet = jnp.transpose(We, (0, 2, 1))

    hsum3 = _sc_scatter_cols(h3, bid_p, SP, z64)
    zcnt = _sc_scatter_small(zuv, bid_p, SP, z16)
    degp = _sc_scatter_small(degvals, dst, SP, z16)

    gs = (32,)
    w_spec = _full((HIDDEN, HIDDEN))
    wet_spec = _full((HIDDEN, 32))
    wc_spec = _full((HIDDEN, 1))
    tbl_out = (jax.ShapeDtypeStruct((SP, D_DST), f32),
               jax.ShapeDtypeStruct((SP, D_SRC), f32),
               jax.ShapeDtypeStruct((SP, D_SRC), f32))
    tbl_spec = (_rows((SB, D_DST)), _rows((SB, D_SRC)), _rows((SB, D_SRC)))

    hb, zc, degv, tdst, tsrca, tsrcc = _tc(
        _k_pool0_body, gs,
        [_mid((NC, SB, 64)), _mid((NC, SB, DS)), _mid((NC, SB, DS)),
         w_spec, w_spec, w_spec, wet_spec, wc_spec],
        (jax.ShapeDtypeStruct((SP, HIDDEN), f32),
         jax.ShapeDtypeStruct((SP, DS), f32),
         jax.ShapeDtypeStruct((SP, DS), f32)) + tbl_out,
        (_rows((SB, HIDDEN)), _rows((SB, DS)), _rows((SB, DS))) + tbl_spec,
    )(hsum3, zcnt, degp, Wq[0], Wk[0], Wv[0], wet[0], Wc[0])

    ge = (EP // RB,)
    hc = hb
    for l in range(LAYERS):
        if l > 0:
            hc, zc, tdst, tsrca, tsrcc = _tc(
                _k_upd_body, gs,
                [_rows((SB, HIDDEN)), _rows((SB, DS)), _mid((NC, SB, 64)),
                 _mid((NC, SB, DS)), _rows((SB, DS)),
                 w_spec, w_spec, w_spec, w_spec, wet_spec, wc_spec],
                (jax.ShapeDtypeStruct((SP, HIDDEN), f32),
                 jax.ShapeDtypeStruct((SP, DS), f32)) + tbl_out,
                (_rows((SB, HIDDEN)), _rows((SB, DS))) + tbl_spec,
            )(hc, zc, agg3, dzp, degv, Wo[l - 1],
              Wq[l], Wk[l], Wv[l], wet[l], Wc[l])

        w16, denp = _sc_pass_a(tdst, tsrca, dst, src, ea_p, z16)
        ddst = _tc(
            _k_ddst_body, gs, [_mid((NC, SB, DS)), _rows((SB, DS))],
            jax.ShapeDtypeStruct((SP, DS), f32), _rows((SB, DS)),
        )(denp, zc)
        gsrcc, gdd = _sc_gather2(tsrcc, src, ddst, dst, D_SRC, DS)
        msg3, dzv = _tc(
            _k_msg_body, ge,
            [_rows((RB, DS)), _rows((RB, DS)), _rows((RB, D_SRC))],
            (jax.ShapeDtypeStruct((NC, EP, 64), f32),
             jax.ShapeDtypeStruct((EP, DS), f32)),
            (_mid((NC, RB, 64)), _rows((RB, DS))),
        )(w16, gdd, gsrcc)
        agg3 = _sc_scatter_cols(msg3, dst, SP, z64)
        dzp = _sc_scatter_small(dzv, dst, SP, z16)

    br, br3 = _tc(
        _k_final_body, gs,
        [_rows((SB, HIDDEN)), _mid((NC, SB, 64)), w_spec],
        (jax.ShapeDtypeStruct((SP, HIDDEN), f32),
         jax.ShapeDtypeStruct((NC, SP, 64), f32)),
        (_rows((SB, HIDDEN)), _mid((NC, SB, 64))),
    )(hc, agg3, Wo[LAYERS - 1])

    grp = _sc_scatter_cols(br3, batch_p, SGP, z64g, ch=112)
    gr = _tc(
        _k_graph_body, (1,), [_full((NC, SGP, 64))],
        jax.ShapeDtypeStruct((SGP, HIDDEN), f32), _full((SGP, HIDDEN)),
    )(grp)

    return (hb[:N_BLOCK], br[:N_BLOCK], gr[:N_GRAPH])


# speedup vs baseline: 1.8922x; 1.0760x over previous
import functools

import jax
import jax.numpy as jnp
import numpy as np
from jax import lax
from jax.experimental import pallas as pl
from jax.experimental.pallas import tpu as pltpu
from jax.experimental.pallas import tpu_sc as plsc

N_UNIT = 100000
N_BLOCK = 25000
N_GRAPH = 64
E = 400000
HIDDEN = 128
NRBF = 16
EDGE = 16
NHEAD = 4
DH = HIDDEN // NHEAD
LAYERS = 3
CUTOFF = 7.0

NC, NS = 2, 16
NW = NC * NS
SP = 25088
EP = 409600
NUP = 102400
SGP = 128
DS = 16
D_DST = 272
D_SRC = 144

_mesh = lambda: plsc.VectorSubcoreMesh(core_axis_name="c", subcore_axis_name="s")


def _sc_gather2(tab1, idx1, tab2, idx2, d1, d2, ch=128):
    n = idx1.shape[0]
    per_w = n // NW
    iters = per_w // ch

    @functools.partial(
        pl.kernel, mesh=_mesh(),
        compiler_params=pltpu.CompilerParams(
            use_tc_tiling_on_sc=False, needs_layout_passes=False),
        out_type=(jax.ShapeDtypeStruct((n, d1), jnp.float32),
                  jax.ShapeDtypeStruct((n, d2), jnp.float32)),
        scratch_types=[pltpu.VMEM((ch,), jnp.int32),
                       pltpu.VMEM((ch,), jnp.int32),
                       pltpu.VMEM((ch, d1), jnp.float32),
                       pltpu.VMEM((ch, d2), jnp.float32),
                       pltpu.SemaphoreType.DMA,
                       pltpu.SemaphoreType.DMA],
    )
    def k(t1, i1, t2, i2, o1, o2, iv1, iv2, r1, r2, s1, s2):
        wid = lax.axis_index("s") * NC + lax.axis_index("c")
        base = wid * per_w

        def body(i, carry):
            b = base + i * ch
            pltpu.sync_copy(i1.at[pl.ds(b, ch)], iv1)
            pltpu.sync_copy(i2.at[pl.ds(b, ch)], iv2)
            c1 = pltpu.async_copy(t1.at[iv1], r1, s1)
            c2 = pltpu.async_copy(t2.at[iv2], r2, s2)
            c1.wait()
            c2.wait()
            pltpu.sync_copy(r1, o1.at[pl.ds(b, ch)])
            pltpu.sync_copy(r2, o2.at[pl.ds(b, ch)])
            return carry

        lax.fori_loop(0, iters, body, 0)

    return k(tab1, idx1, tab2, idx2)


def _sc_gather1(tab, idx, d, ch=128):
    n = idx.shape[0]
    per_w = n // NW
    iters = per_w // ch

    @functools.partial(
        pl.kernel, mesh=_mesh(),
        compiler_params=pltpu.CompilerParams(
            use_tc_tiling_on_sc=False, needs_layout_passes=False),
        out_type=jax.ShapeDtypeStruct((n, d), jnp.float32),
        scratch_types=[pltpu.VMEM((ch,), jnp.int32),
                       pltpu.VMEM((ch, d), jnp.float32),
                       pltpu.SemaphoreType.DMA],
    )
    def k(t1, i1, o1, iv1, r1, s1):
        wid = lax.axis_index("s") * NC + lax.axis_index("c")
        base = wid * per_w

        def body(i, carry):
            b = base + i * ch
            pltpu.sync_copy(i1.at[pl.ds(b, ch)], iv1)
            pltpu.async_copy(t1.at[iv1], r1, s1).wait()
            pltpu.sync_copy(r1, o1.at[pl.ds(b, ch)])
            return carry

        lax.fori_loop(0, iters, body, 0)

    return k(tab, idx)


def _sc_scatter_cols(vals3, idx, s_out, zeros64, ch=128):
    n = idx.shape[0]
    per_t = n // NS
    iters = per_t // ch
    stripe = s_out // NS

    @functools.partial(
        pl.kernel, mesh=_mesh(),
        compiler_params=pltpu.CompilerParams(
            use_tc_tiling_on_sc=False, needs_layout_passes=False),
        out_type=jax.ShapeDtypeStruct((NC, s_out, 64), jnp.float32),
        scratch_types=[pltpu.VMEM((ch,), jnp.int32),
                       pltpu.VMEM((ch, 64), jnp.float32),
                       pltpu.VMEM_SHARED((s_out, 64), jnp.float32)],
    )
    def k(v_hbm, i_hbm, z_hbm, o_hbm, iv, vv, acc):
        cid = lax.axis_index("c")
        sid = lax.axis_index("s")
        r0 = sid * stripe
        pltpu.sync_copy(z_hbm.at[pl.ds(r0, stripe)], acc.at[pl.ds(r0, stripe)])
        plsc.subcore_barrier()

        def body(i, carry):
            b = sid * per_t + i * ch
            pltpu.sync_copy(i_hbm.at[pl.ds(b, ch)], iv)
            pltpu.sync_copy(v_hbm.at[cid, pl.ds(b, ch), :], vv)
            pltpu.sync_copy(vv, acc.at[iv], add=True)
            return carry

        lax.fori_loop(0, iters, body, 0)
        plsc.subcore_barrier()
        pltpu.sync_copy(acc.at[pl.ds(r0, stripe)],
                        o_hbm.at[cid, pl.ds(r0, stripe), :])

    return k(vals3, idx, zeros64)


def _sc_scatter_small(vals, idx, s_out, zeros16, ch=128, w=DS):
    n = idx.shape[0]
    per_w = n // NW
    iters = per_w // ch
    stripe = s_out // NS

    @functools.partial(
        pl.kernel, mesh=_mesh(),
        compiler_params=pltpu.CompilerParams(
            use_tc_tiling_on_sc=False, needs_layout_passes=False),
        out_type=jax.ShapeDtypeStruct((NC, s_out, w), jnp.float32),
        scratch_types=[pltpu.VMEM((ch,), jnp.int32),
                       pltpu.VMEM((ch, w), jnp.float32),
                       pltpu.VMEM_SHARED((s_out, w), jnp.float32)],
    )
    def k(v_hbm, i_hbm, z_hbm, o_hbm, iv, vv, acc):
        cid = lax.axis_index("c")
        sid = lax.axis_index("s")
        wid = sid * NC + cid
        r0 = sid * stripe
        pltpu.sync_copy(z_hbm.at[pl.ds(r0, stripe)], acc.at[pl.ds(r0, stripe)])
        plsc.subcore_barrier()

        def body(i, carry):
            b = wid * per_w + i * ch
            pltpu.sync_copy(i_hbm.at[pl.ds(b, ch)], iv)
            pltpu.sync_copy(v_hbm.at[pl.ds(b, ch), :], vv)
            pltpu.sync_copy(vv, acc.at[iv], add=True)
            return carry

        lax.fori_loop(0, iters, body, 0)
        plsc.subcore_barrier()
        pltpu.sync_copy(acc.at[pl.ds(r0, stripe)],
                        o_hbm.at[cid, pl.ds(r0, stripe), :])

    return k(vals, idx, zeros16)


def _vsqrt(r2):
    ib = plsc.bitcast(r2, jnp.int32)
    x = plsc.bitcast((ib >> 1) + jnp.int32(0x1FBD1DF5), jnp.float32)
    for _ in range(3):
        x = 0.5 * (x + r2 / x)
    return x


def _sc_pass_a(tdst, tsrca, dst, src, ea, z16, ch=128):
    per_w = EP // NW
    iters = per_w // ch
    stripe = SP // NS
    isq = float(1.0 / np.sqrt(DH))
    inv2s = float(1.0 / _RBF_DEN)
    cstep = float(CUTOFF / (NRBF - 1))

    @functools.partial(
        pl.kernel, mesh=_mesh(),
        compiler_params=pltpu.CompilerParams(
            use_tc_tiling_on_sc=False, needs_layout_passes=False),
        out_type=(jax.ShapeDtypeStruct((EP, DS), jnp.float32),
                  jax.ShapeDtypeStruct((NC, SP, DS), jnp.float32)),
        scratch_types=[pltpu.VMEM((ch,), jnp.int32),
                       pltpu.VMEM((ch,), jnp.int32),
                       pltpu.VMEM((ch, D_DST), jnp.float32),
                       pltpu.VMEM((ch, D_SRC), jnp.float32),
                       pltpu.VMEM((ch, EDGE), jnp.float32),
                       pltpu.VMEM((ch, DS), jnp.float32),
                       pltpu.VMEM_SHARED((SP, DS), jnp.float32),
                       pltpu.SemaphoreType.DMA,
                       pltpu.SemaphoreType.DMA],
    )
    def k(td_h, ts_h, di_h, si_h, ea_h, z_h, w_h, dp_h,
          dvi, svi, gd, gs, eab, wb, dacc, s1, s2):
        cid = lax.axis_index("c")
        sid = lax.axis_index("s")
        wid = sid * NC + cid
        r0 = sid * stripe
        pltpu.sync_copy(z_h.at[pl.ds(r0, stripe)], dacc.at[pl.ds(r0, stripe)])
        pltpu.sync_copy(z_h.at[pl.ds(0, ch)], wb)
        plsc.subcore_barrier()
        lanes = lax.iota(jnp.int32, 16)

        def chunk(i, carry):
            b = wid * per_w + i * ch
            pltpu.sync_copy(di_h.at[pl.ds(b, ch)], dvi)
            pltpu.sync_copy(si_h.at[pl.ds(b, ch)], svi)
            c1 = pltpu.async_copy(td_h.at[dvi], gd, s1)
            c2 = pltpu.async_copy(ts_h.at[svi], gs, s2)
            c1.wait()
            c2.wait()
            pltpu.sync_copy(ea_h.at[pl.ds(b, ch), :], eab)

            def group(g, c3):
                rows = g * 16 + lanes

                def col(ref, c):
                    return plsc.load_gather(
                        ref, [rows, jnp.full((16,), c, jnp.int32)])

                rel2 = jnp.full((16,), 1e-8, jnp.float32)
                for t in range(3):
                    dt = col(gd, 256 + t) - col(gs, 128 + t)
                    rel2 = rel2 + dt * dt
                d = _vsqrt(rel2)
                feat = []
                for j in range(NRBF):
                    t = d - (cstep * j)
                    feat.append(jnp.exp(-(t * t) * inv2s))
                for j in range(EDGE):
                    feat.append(col(eab, j))
                valid = (b + rows) < E
                for h in range(NHEAD):
                    acc = jnp.zeros((16,), jnp.float32)
                    for c in range(DH):
                        acc = acc + col(gd, 32 * h + c) * col(gs, 32 * h + c)
                    for j in range(32):
                        acc = acc + feat[j] * col(gd, 128 + 32 * h + j)
                    w = jnp.exp(acc * isq)
                    w = jnp.where(valid, w, 0.0)
                    plsc.store_scatter(
                        wb, [rows, jnp.full((16,), h, jnp.int32)], w)
                return c3

            lax.fori_loop(0, ch // 16, group, 0)
            pltpu.sync_copy(wb, w_h.at[pl.ds(b, ch)])
            pltpu.sync_copy(wb, dacc.at[dvi], add=True)
            return carry

        lax.fori_loop(0, iters, chunk, 0)
        plsc.subcore_barrier()
        pltpu.sync_copy(dacc.at[pl.ds(r0, stripe)],
                        dp_h.at[cid, pl.ds(r0, stripe), :])

    return k(tdst, tsrca, dst, src, ea, z16)


def _sc_pass_c(tcat, ddst, w16, dst, src2, z64, z8, ch=128):
    per_t = EP // NS
    iters = per_t // ch
    stripe = SP // NS

    @functools.partial(
        pl.kernel, mesh=_mesh(),
        compiler_params=pltpu.CompilerParams(
            use_tc_tiling_on_sc=False, needs_layout_passes=False),
        out_type=(jax.ShapeDtypeStruct((NC, SP, 64), jnp.float32),
                  jax.ShapeDtypeStruct((EP, 8), jnp.float32)),
        scratch_types=[pltpu.VMEM((ch,), jnp.int32),
                       pltpu.VMEM((ch,), jnp.int32),
                       pltpu.VMEM((ch, 80), jnp.float32),
                       pltpu.VMEM((ch, DS), jnp.float32),
                       pltpu.VMEM((ch, DS), jnp.float32),
                       pltpu.VMEM((ch, 64), jnp.float32),
                       pltpu.VMEM((ch, 8), jnp.float32),
                       pltpu.VMEM_SHARED((SP, 64), jnp.float32),
                       pltpu.SemaphoreType.DMA,
                       pltpu.SemaphoreType.DMA],
    )
    def k(tc_h, dd_h, w_h, di_h, si_h, z64_h, z8_h, agg_h, dz_h,
          dvi, svi, gv, gdd, wbuf, mbuf, dzbuf, acc64, s1, s2):
        cid = lax.axis_index("c")
        sid = lax.axis_index("s")
        r0 = sid * stripe
        pltpu.sync_copy(z64_h.at[pl.ds(r0, stripe)], acc64.at[pl.ds(r0, stripe)])
        pltpu.sync_copy(z8_h.at[pl.ds(0, ch)], dzbuf)
        plsc.subcore_barrier()
        lanes = lax.iota(jnp.int32, 16)

        def chunk(i, carry):
            b = sid * per_t + i * ch
            pltpu.sync_copy(di_h.at[pl.ds(b, ch)], dvi)
            pltpu.sync_copy(si_h.at[cid, pl.ds(b, ch)], svi)
            c1 = pltpu.async_copy(tc_h.at[svi], gv, s1)
            c2 = pltpu.async_copy(dd_h.at[dvi], gdd, s2)
            c1.wait()
            c2.wait()
            pltpu.sync_copy(w_h.at[pl.ds(b, ch), :], wbuf)

            def group(g, c3):
                rows = g * 16 + lanes

                def col(ref, c):
                    return plsc.load_gather(
                        ref, [rows, jnp.full((16,), c, jnp.int32)])

                a = []
                for h in range(NHEAD):
                    a.append(col(wbuf, h) / col(gdd, h))
                is0 = (cid == 0)
                a_lo = jnp.where(is0, a[0], a[2])
                a_hi = jnp.where(is0, a[1], a[3])
                for c in range(64):
                    av = a_lo if c < 32 else a_hi
                    m = av * col(gv, c)
                    plsc.store_scatter(
                        mbuf, [rows, jnp.full((16,), c, jnp.int32)], m)
                csc = jnp.zeros((16,), jnp.float32)
                for h in range(NHEAD):
                    csc = csc + a[h] * col(gv, 64 + h)
                for t in range(3):
                    dz = (col(gdd, NHEAD + t) - col(gv, 68 + t)) * csc
                    plsc.store_scatter(
                        dzbuf, [rows, jnp.full((16,), t, jnp.int32)], dz)
                return c3

            lax.fori_loop(0, ch // 16, group, 0)
            pltpu.sync_copy(mbuf, acc64.at[dvi], add=True)

            @pl.when(cid == 0)
            def _():
                pltpu.sync_copy(dzbuf, dz_h.at[pl.ds(b, ch)])

            return carry

        lax.fori_loop(0, iters, chunk, 0)
        plsc.subcore_barrier()
        pltpu.sync_copy(acc64.at[pl.ds(r0, stripe)],
                        agg_h.at[cid, pl.ds(r0, stripe), :])

    return k(tcat, ddst, w16, dst, src2, z64, z8)


def _tc(body, grid, in_specs, out_shapes, out_specs):
    return pl.pallas_call(
        body, grid=grid, in_specs=in_specs,
        out_shape=out_shapes, out_specs=out_specs)


def _full(shape):
    return pl.BlockSpec(shape, lambda i: tuple(0 for _ in shape))


def _rows(shape):
    nd = len(shape)
    return pl.BlockSpec(shape, lambda i: (i,) + (0,) * (nd - 1))


def _mid(shape):
    return pl.BlockSpec(shape, lambda i: (0, i, 0))


SB = 784
RB = 2048


def _tables(hb, zc, wq, wk, wv, wet, wc, tdst_ref, tsrca_ref, tsrcc_ref):
    q = jnp.dot(hb, wq, preferred_element_type=jnp.float32)
    kk = jnp.dot(hb, wk, preferred_element_type=jnp.float32)
    v = jnp.dot(hb, wv, preferred_element_type=jnp.float32)
    a = jnp.concatenate(
        [jnp.dot(q[:, 32 * h:32 * h + 32], wet[32 * h:32 * h + 32, :],
                 preferred_element_type=jnp.float32) for h in range(NHEAD)],
        axis=1)
    vc = jnp.sum((v * wc[:, 0][None, :]).reshape(-1, NHEAD, DH), axis=-1)
    z3 = zc[:, 0:3]
    zpad = jnp.zeros((hb.shape[0], D_DST - 259), jnp.float32)
    tdst_ref[...] = jnp.concatenate([q, a, z3, zpad], axis=1)
    spad = jnp.zeros((hb.shape[0], D_SRC - 131), jnp.float32)
    tsrca_ref[...] = jnp.concatenate([kk, z3, spad], axis=1)
    cpad = jnp.zeros((hb.shape[0], 80 - 71), jnp.float32)
    h0 = jnp.concatenate([v[:, 0:64], vc[:, 0:2], vc[:, 2:4], z3, cpad], axis=1)
    h1 = jnp.concatenate([v[:, 64:128], vc[:, 0:2], vc[:, 2:4], z3, cpad], axis=1)
    tsrcc_ref[...] = jnp.stack([h0, h1], axis=0)


def _k_pool0_body(hs_ref, zcnt_ref, degp_ref, wq_ref, wk_ref, wv_ref,
                  wet_ref, wc_ref, hb_ref, zc_ref, degv_ref,
                  tdst_ref, tsrca_ref, tsrcc_ref):
    zc01 = zcnt_ref[0] + zcnt_ref[1]
    cnt = jnp.maximum(zc01[:, 3:4], 1.0)
    hb = jnp.concatenate([hs_ref[0], hs_ref[1]], axis=1) / cnt
    z3 = zc01[:, 0:3] / cnt
    deg01 = degp_ref[0] + degp_ref[1]
    deg = jnp.maximum(deg01[:, 0:1], 1.0)
    pad13 = jnp.zeros((SB, DS - 3), jnp.float32)
    zc = jnp.concatenate([z3, pad13], axis=1)
    hb_ref[...] = hb
    zc_ref[...] = zc
    pad15 = jnp.zeros((SB, DS - 1), jnp.float32)
    degv_ref[...] = jnp.concatenate([deg, pad15], axis=1)
    _tables(hb, zc, wq_ref[...], wk_ref[...], wv_ref[...], wet_ref[...],
            wc_ref[...], tdst_ref, tsrca_ref, tsrcc_ref)


def _k_upd_body(hc_ref, zc_ref, agg_ref, dzp_ref, degv_ref, wo_ref,
                wq_ref, wk_ref, wv_ref, wet_ref, wc_ref,
                hcn_ref, zcn_ref, tdst_ref, tsrca_ref, tsrcc_ref):
    agg = jnp.concatenate([agg_ref[0], agg_ref[1]], axis=1)
    hc = hc_ref[...] + jnp.dot(agg, wo_ref[...],
                               preferred_element_type=jnp.float32)
    dz01 = dzp_ref[0] + dzp_ref[1]
    deg = degv_ref[...][:, 0:1]
    z3 = zc_ref[...][:, 0:3] + dz01[:, 0:3] / deg
    pad13 = jnp.zeros((SB, DS - 3), jnp.float32)
    zc = jnp.concatenate([z3, pad13], axis=1)
    hcn_ref[...] = hc
    zcn_ref[...] = zc
    _tables(hc, zc, wq_ref[...], wk_ref[...], wv_ref[...], wet_ref[...],
            wc_ref[...], tdst_ref, tsrca_ref, tsrcc_ref)


_RBF_DEN = 2.0 * (CUTOFF / NRBF) ** 2
_ISQ = 1.0 / np.sqrt(DH)


def _k_logits_body(gd_ref, gs_ref, ea_ref, lo_ref):
    i = pl.program_id(0)
    gd = gd_ref[...]
    gs = gs_ref[...]
    rel = gd[:, 256:259] - gs[:, 128:131]
    d = jnp.sqrt(jnp.sum(rel * rel, axis=1, keepdims=True) + 1e-8)
    centers = (lax.broadcasted_iota(jnp.int32, (1, NRBF), 1).astype(jnp.float32)
               * (CUTOFF / (NRBF - 1)))
    rbf = jnp.exp(-((d - centers) ** 2) / _RBF_DEN)
    feat = jnp.concatenate([rbf, ea_ref[...]], axis=1)
    qk = jnp.sum((gd[:, 0:128] * gs[:, 0:128]).reshape(RB, NHEAD, DH), axis=-1)
    fa = jnp.sum(gd[:, 128:256].reshape(RB, NHEAD, DH) * feat[:, None, :],
                 axis=-1)
    w = jnp.exp((qk + fa) * _ISQ)
    rows = i * RB + lax.broadcasted_iota(jnp.int32, (RB, 1), 0)
    w = jnp.where(rows < E, w, 0.0)
    lo_ref[...] = jnp.concatenate(
        [w, jnp.zeros((RB, DS - NHEAD), jnp.float32)], axis=1)


def _k_ctab_body(lsum_ref, degv_ref, ct_ref):
    lsum = lsum_ref[0] + lsum_ref[1]
    c = lsum[:, 0:NHEAD] / degv_ref[...][:, 0:1]
    ct_ref[...] = jnp.concatenate(
        [c, jnp.zeros((SB, DS - NHEAD), jnp.float32)], axis=1)


def _k_w_body(lo_ref, gc_ref, w_ref):
    i = pl.program_id(0)
    w = jnp.exp(lo_ref[...][:, 0:NHEAD] - gc_ref[...][:, 0:NHEAD])
    rows = i * RB + lax.broadcasted_iota(jnp.int32, (RB, 1), 0)
    w = jnp.where(rows < E, w, 0.0)
    w_ref[...] = jnp.concatenate(
        [w, jnp.zeros((RB, DS - NHEAD), jnp.float32)], axis=1)


def _k_ddst_body(denp_ref, zc_ref, dd_ref):
    den = denp_ref[0] + denp_ref[1]
    dd = den[:, 0:NHEAD] + 1e-8
    z3 = zc_ref[...][:, 0:3]
    dd_ref[...] = jnp.concatenate(
        [dd, z3, jnp.zeros((SB, DS - NHEAD - 3), jnp.float32)], axis=1)


def _k_msg_body(w_ref, gdd_ref, gs_ref, msg_ref, dzv_ref):
    gdd = gdd_ref[...]
    gs = gs_ref[...]
    alpha = w_ref[...][:, 0:NHEAD] / gdd[:, 0:NHEAD]
    v = gs[:, 0:128].reshape(RB, NHEAD, DH)
    msg = (alpha[:, :, None] * v).reshape(RB, HIDDEN)
    msg_ref[...] = jnp.stack([msg[:, 0:64], msg[:, 64:128]], axis=0)
    csc = jnp.sum(alpha * gs[:, 128:132], axis=1, keepdims=True)
    rel = gdd[:, NHEAD:NHEAD + 3] - gs[:, 132:135]
    dzv_ref[...] = jnp.concatenate(
        [rel * csc, jnp.zeros((RB, DS - 3), jnp.float32)], axis=1)


def _k_final_body(hc_ref, agg_ref, wo_ref, br_ref, br3_ref):
    agg = jnp.concatenate([agg_ref[0], agg_ref[1]], axis=1)
    hc = hc_ref[...] + jnp.dot(agg, wo_ref[...],
                               preferred_element_type=jnp.float32)
    nrm = jnp.sqrt(jnp.sum(hc * hc, axis=1, keepdims=True))
    br = hc / (nrm + 1e-12)
    br_ref[...] = br
    br3_ref[...] = jnp.stack([br[:, 0:64], br[:, 64:128]], axis=0)


def _k_graph_body(grp_ref, gr_ref):
    g = jnp.concatenate([grp_ref[0], grp_ref[1]], axis=1)
    nrm = jnp.sqrt(jnp.sum(g * g, axis=1, keepdims=True))
    gr_ref[...] = g / (nrm + 1e-12)


def kernel(H, Z, block_id, batch_id, edges, edge_attr, Wq, Wk, Wv, We, Wo, Wc):
    f32 = jnp.float32
    src = jnp.pad(edges[0].astype(jnp.int32), (0, EP - E))
    dst = jnp.pad(edges[1].astype(jnp.int32), (0, EP - E))
    ea_p = jnp.pad(edge_attr, ((0, EP - E), (0, 0)))
    emask = (jnp.arange(EP) < E).astype(f32)[:, None]
    degvals = jnp.pad(emask, ((0, 0), (0, DS - 1)))
    bid_p = jnp.pad(block_id.astype(jnp.int32), (0, NUP - N_UNIT))
    hp = jnp.pad(H, ((0, NUP - N_UNIT), (0, 0)))
    h3 = jnp.stack([hp[:, 0:64], hp[:, 64:128]], axis=0)
    zuv = jnp.pad(
        jnp.concatenate([Z.reshape(N_UNIT, 3), jnp.ones((N_UNIT, 1), f32)],
                        axis=1),
        ((0, NUP - N_UNIT), (0, DS - 4)))
    batch_p = jnp.pad(batch_id.astype(jnp.int32), (0, SP - N_BLOCK))
    z64 = jnp.zeros((SP, 64), f32)
    z64g = jnp.zeros((SGP, 64), f32)
    z16 = jnp.zeros((SP, DS), f32)
    z8 = jnp.zeros((SP, 8), f32)
    src2 = jnp.stack([src, src + SP], axis=0)
    wet = jnp.transpose(We, (0, 2, 1))

    hsum3 = _sc_scatter_cols(h3, bid_p, SP, z64)
    zcnt = _sc_scatter_small(zuv, bid_p, SP, z16)
    degp = _sc_scatter_small(degvals, dst, SP, z16)

    gs = (32,)
    w_spec = _full((HIDDEN, HIDDEN))
    wet_spec = _full((HIDDEN, 32))
    wc_spec = _full((HIDDEN, 1))
    tbl_out = (jax.ShapeDtypeStruct((SP, D_DST), f32),
               jax.ShapeDtypeStruct((SP, D_SRC), f32),
               jax.ShapeDtypeStruct((NC, SP, 80), f32))
    tbl_spec = (_rows((SB, D_DST)), _rows((SB, D_SRC)), _mid((NC, SB, 80)))

    hb, zc, degv, tdst, tsrca, tsrcc = _tc(
        _k_pool0_body, gs,
        [_mid((NC, SB, 64)), _mid((NC, SB, DS)), _mid((NC, SB, DS)),
         w_spec, w_spec, w_spec, wet_spec, wc_spec],
        (jax.ShapeDtypeStruct((SP, HIDDEN), f32),
         jax.ShapeDtypeStruct((SP, DS), f32),
         jax.ShapeDtypeStruct((SP, DS), f32)) + tbl_out,
        (_rows((SB, HIDDEN)), _rows((SB, DS)), _rows((SB, DS))) + tbl_spec,
    )(hsum3, zcnt, degp, Wq[0], Wk[0], Wv[0], wet[0], Wc[0])

    ge = (EP // RB,)
    hc = hb
    for l in range(LAYERS):
        if l > 0:
            hc, zc, tdst, tsrca, tsrcc = _tc(
                _k_upd_body, gs,
                [_rows((SB, HIDDEN)), _rows((SB, DS)), _mid((NC, SB, 64)),
                 _mid((NC, SB, 8)), _rows((SB, DS)),
                 w_spec, w_spec, w_spec, w_spec, wet_spec, wc_spec],
                (jax.ShapeDtypeStruct((SP, HIDDEN), f32),
                 jax.ShapeDtypeStruct((SP, DS), f32)) + tbl_out,
                (_rows((SB, HIDDEN)), _rows((SB, DS))) + tbl_spec,
            )(hc, zc, agg3, dzp, degv, Wo[l - 1],
              Wq[l], Wk[l], Wv[l], wet[l], Wc[l])

        w16, denp = _sc_pass_a(tdst, tsrca, dst, src, ea_p, z16)
        ddst = _tc(
            _k_ddst_body, gs, [_mid((NC, SB, DS)), _rows((SB, DS))],
            jax.ShapeDtypeStruct((SP, DS), f32), _rows((SB, DS)),
        )(denp, zc)
        tcat = tsrcc.reshape(NC * SP, 80)
        agg3, dzv = _sc_pass_c(tcat, ddst, w16, dst, src2, z64, z8)
        dzp = _sc_scatter_small(dzv, dst, SP, z8, w=8)

    br, br3 = _tc(
        _k_final_body, gs,
        [_rows((SB, HIDDEN)), _mid((NC, SB, 64)), w_spec],
        (jax.ShapeDtypeStruct((SP, HIDDEN), f32),
         jax.ShapeDtypeStruct((NC, SP, 64), f32)),
        (_rows((SB, HIDDEN)), _mid((NC, SB, 64))),
    )(hc, agg3, Wo[LAYERS - 1])

    grp = _sc_scatter_cols(br3, batch_p, SGP, z64g, ch=112)
    gr = _tc(
        _k_graph_body, (1,), [_full((NC, SGP, 64))],
        jax.ShapeDtypeStruct((SGP, HIDDEN), f32), _full((SGP, HIDDEN)),
    )(grp)

    return (hb[:N_BLOCK], br[:N_BLOCK], gr[:N_GRAPH])


# speedup vs baseline: 2.0293x; 1.0725x over previous
import functools

import jax
import jax.numpy as jnp
import numpy as np
from jax import lax
from jax.experimental import pallas as pl
from jax.experimental.pallas import tpu as pltpu
from jax.experimental.pallas import tpu_sc as plsc

N_UNIT = 100000
N_BLOCK = 25000
N_GRAPH = 64
E = 400000
HIDDEN = 128
NRBF = 16
EDGE = 16
NHEAD = 4
DH = HIDDEN // NHEAD
LAYERS = 3
CUTOFF = 7.0

NC, NS = 2, 16
NW = NC * NS
SP = 25088
EP = 409600
NUP = 102400
SGP = 128
DS = 16
D_DST = 272
D_SRC = 144

_mesh = lambda: plsc.VectorSubcoreMesh(core_axis_name="c", subcore_axis_name="s")


def _sc_gather2(tab1, idx1, tab2, idx2, d1, d2, ch=128):
    n = idx1.shape[0]
    per_w = n // NW
    iters = per_w // ch

    @functools.partial(
        pl.kernel, mesh=_mesh(),
        compiler_params=pltpu.CompilerParams(
            use_tc_tiling_on_sc=False, needs_layout_passes=False),
        out_type=(jax.ShapeDtypeStruct((n, d1), jnp.float32),
                  jax.ShapeDtypeStruct((n, d2), jnp.float32)),
        scratch_types=[pltpu.VMEM((ch,), jnp.int32),
                       pltpu.VMEM((ch,), jnp.int32),
                       pltpu.VMEM((ch, d1), jnp.float32),
                       pltpu.VMEM((ch, d2), jnp.float32),
                       pltpu.SemaphoreType.DMA,
                       pltpu.SemaphoreType.DMA],
    )
    def k(t1, i1, t2, i2, o1, o2, iv1, iv2, r1, r2, s1, s2):
        wid = lax.axis_index("s") * NC + lax.axis_index("c")
        base = wid * per_w

        def body(i, carry):
            b = base + i * ch
            pltpu.sync_copy(i1.at[pl.ds(b, ch)], iv1)
            pltpu.sync_copy(i2.at[pl.ds(b, ch)], iv2)
            c1 = pltpu.async_copy(t1.at[iv1], r1, s1)
            c2 = pltpu.async_copy(t2.at[iv2], r2, s2)
            c1.wait()
            c2.wait()
            pltpu.sync_copy(r1, o1.at[pl.ds(b, ch)])
            pltpu.sync_copy(r2, o2.at[pl.ds(b, ch)])
            return carry

        lax.fori_loop(0, iters, body, 0)

    return k(tab1, idx1, tab2, idx2)


def _sc_gather1(tab, idx, d, ch=128):
    n = idx.shape[0]
    per_w = n // NW
    iters = per_w // ch

    @functools.partial(
        pl.kernel, mesh=_mesh(),
        compiler_params=pltpu.CompilerParams(
            use_tc_tiling_on_sc=False, needs_layout_passes=False),
        out_type=jax.ShapeDtypeStruct((n, d), jnp.float32),
        scratch_types=[pltpu.VMEM((ch,), jnp.int32),
                       pltpu.VMEM((ch, d), jnp.float32),
                       pltpu.SemaphoreType.DMA],
    )
    def k(t1, i1, o1, iv1, r1, s1):
        wid = lax.axis_index("s") * NC + lax.axis_index("c")
        base = wid * per_w

        def body(i, carry):
            b = base + i * ch
            pltpu.sync_copy(i1.at[pl.ds(b, ch)], iv1)
            pltpu.async_copy(t1.at[iv1], r1, s1).wait()
            pltpu.sync_copy(r1, o1.at[pl.ds(b, ch)])
            return carry

        lax.fori_loop(0, iters, body, 0)

    return k(tab, idx)


def _sc_scatter_cols(vals3, idx, s_out, zeros64, ch=128):
    n = idx.shape[0]
    per_t = n // NS
    iters = per_t // ch
    stripe = s_out // NS

    @functools.partial(
        pl.kernel, mesh=_mesh(),
        compiler_params=pltpu.CompilerParams(
            use_tc_tiling_on_sc=False, needs_layout_passes=False),
        out_type=jax.ShapeDtypeStruct((NC, s_out, 64), jnp.float32),
        scratch_types=[pltpu.VMEM((ch,), jnp.int32),
                       pltpu.VMEM((ch, 64), jnp.float32),
                       pltpu.VMEM_SHARED((s_out, 64), jnp.float32)],
    )
    def k(v_hbm, i_hbm, z_hbm, o_hbm, iv, vv, acc):
        cid = lax.axis_index("c")
        sid = lax.axis_index("s")
        r0 = sid * stripe
        pltpu.sync_copy(z_hbm.at[pl.ds(r0, stripe)], acc.at[pl.ds(r0, stripe)])
        plsc.subcore_barrier()

        def body(i, carry):
            b = sid * per_t + i * ch
            pltpu.sync_copy(i_hbm.at[pl.ds(b, ch)], iv)
            pltpu.sync_copy(v_hbm.at[cid, pl.ds(b, ch), :], vv)
            pltpu.sync_copy(vv, acc.at[iv], add=True)
            return carry

        lax.fori_loop(0, iters, body, 0)
        plsc.subcore_barrier()
        pltpu.sync_copy(acc.at[pl.ds(r0, stripe)],
                        o_hbm.at[cid, pl.ds(r0, stripe), :])

    return k(vals3, idx, zeros64)


def _sc_scatter_small(vals, idx, s_out, zeros16, ch=128, w=DS):
    n = idx.shape[0]
    per_w = n // NW
    iters = per_w // ch
    stripe = s_out // NS

    @functools.partial(
        pl.kernel, mesh=_mesh(),
        compiler_params=pltpu.CompilerParams(
            use_tc_tiling_on_sc=False, needs_layout_passes=False),
        out_type=jax.ShapeDtypeStruct((NC, s_out, w), jnp.float32),
        scratch_types=[pltpu.VMEM((ch,), jnp.int32),
                       pltpu.VMEM((ch, w), jnp.float32),
                       pltpu.VMEM_SHARED((s_out, w), jnp.float32)],
    )
    def k(v_hbm, i_hbm, z_hbm, o_hbm, iv, vv, acc):
        cid = lax.axis_index("c")
        sid = lax.axis_index("s")
        wid = sid * NC + cid
        r0 = sid * stripe
        pltpu.sync_copy(z_hbm.at[pl.ds(r0, stripe)], acc.at[pl.ds(r0, stripe)])
        plsc.subcore_barrier()

        def body(i, carry):
            b = wid * per_w + i * ch
            pltpu.sync_copy(i_hbm.at[pl.ds(b, ch)], iv)
            pltpu.sync_copy(v_hbm.at[pl.ds(b, ch), :], vv)
            pltpu.sync_copy(vv, acc.at[iv], add=True)
            return carry

        lax.fori_loop(0, iters, body, 0)
        plsc.subcore_barrier()
        pltpu.sync_copy(acc.at[pl.ds(r0, stripe)],
                        o_hbm.at[cid, pl.ds(r0, stripe), :])

    return k(vals, idx, zeros16)


def _vsqrt(r2):
    ib = plsc.bitcast(r2, jnp.int32)
    x = plsc.bitcast((ib >> 1) + jnp.int32(0x1FBD1DF5), jnp.float32)
    for _ in range(3):
        x = 0.5 * (x + r2 / x)
    return x


def _sc_pass_a(tdst, tsrca, dst, src, ea, z16, ch=128):
    per_w = EP // NW
    iters = per_w // ch
    stripe = SP // NS
    isq = float(1.0 / np.sqrt(DH))
    inv2s = float(1.0 / _RBF_DEN)
    cstep = float(CUTOFF / (NRBF - 1))

    @functools.partial(
        pl.kernel, mesh=_mesh(),
        compiler_params=pltpu.CompilerParams(
            use_tc_tiling_on_sc=False, needs_layout_passes=False),
        out_type=(jax.ShapeDtypeStruct((EP, DS), jnp.float32),
                  jax.ShapeDtypeStruct((NC, SP, DS), jnp.float32)),
        scratch_types=[pltpu.VMEM((ch,), jnp.int32),
                       pltpu.VMEM((ch,), jnp.int32),
                       pltpu.VMEM((ch, D_DST), jnp.float32),
                       pltpu.VMEM((ch, D_SRC), jnp.float32),
                       pltpu.VMEM((ch, EDGE), jnp.float32),
                       pltpu.VMEM((ch, DS), jnp.float32),
                       pltpu.VMEM_SHARED((SP, DS), jnp.float32),
                       pltpu.SemaphoreType.DMA,
                       pltpu.SemaphoreType.DMA],
    )
    def k(td_h, ts_h, di_h, si_h, ea_h, z_h, w_h, dp_h,
          dvi, svi, gd, gs, eab, wb, dacc, s1, s2):
        cid = lax.axis_index("c")
        sid = lax.axis_index("s")
        wid = sid * NC + cid
        r0 = sid * stripe
        pltpu.sync_copy(z_h.at[pl.ds(r0, stripe)], dacc.at[pl.ds(r0, stripe)])
        pltpu.sync_copy(z_h.at[pl.ds(0, ch)], wb)
        plsc.subcore_barrier()
        lanes = lax.iota(jnp.int32, 16)

        def chunk(i, carry):
            b = wid * per_w + i * ch
            pltpu.sync_copy(di_h.at[pl.ds(b, ch)], dvi)
            pltpu.sync_copy(si_h.at[pl.ds(b, ch)], svi)
            c1 = pltpu.async_copy(td_h.at[dvi], gd, s1)
            c2 = pltpu.async_copy(ts_h.at[svi], gs, s2)
            c1.wait()
            c2.wait()
            pltpu.sync_copy(ea_h.at[pl.ds(b, ch), :], eab)

            def group(g, c3):
                rows = g * 16 + lanes

                def col(ref, c):
                    return plsc.load_gather(
                        ref, [rows, jnp.full((16,), c, jnp.int32)])

                rel2 = jnp.full((16,), 1e-8, jnp.float32)
                for t in range(3):
                    dt = col(gd, 256 + t) - col(gs, 128 + t)
                    rel2 = rel2 + dt * dt
                d = _vsqrt(rel2)
                feat = []
                for j in range(NRBF):
                    t = d - (cstep * j)
                    feat.append(jnp.exp(-(t * t) * inv2s))
                for j in range(EDGE):
                    feat.append(col(eab, j))
                valid = (b + rows) < E
                for h in range(NHEAD):
                    acc = jnp.zeros((16,), jnp.float32)
                    for c in range(DH):
                        acc = acc + col(gd, 32 * h + c) * col(gs, 32 * h + c)
                    for j in range(32):
                        acc = acc + feat[j] * col(gd, 128 + 32 * h + j)
                    w = jnp.exp(acc * isq)
                    w = jnp.where(valid, w, 0.0)
                    plsc.store_scatter(
                        wb, [rows, jnp.full((16,), h, jnp.int32)], w)
                return c3

            lax.fori_loop(0, ch // 16, group, 0)
            pltpu.sync_copy(wb, w_h.at[pl.ds(b, ch)])
            pltpu.sync_copy(wb, dacc.at[dvi], add=True)
            return carry

        lax.fori_loop(0, iters, chunk, 0)
        plsc.subcore_barrier()
        pltpu.sync_copy(dacc.at[pl.ds(r0, stripe)],
                        dp_h.at[cid, pl.ds(r0, stripe), :])

    return k(tdst, tsrca, dst, src, ea, z16)


def _sc_pass_c(tcat, ddst, w16, dst, src2, z64, z8, ch=80):
    per_t = EP // NS
    iters = per_t // ch
    stripe = SP // NS

    @functools.partial(
        pl.kernel, mesh=_mesh(),
        compiler_params=pltpu.CompilerParams(
            use_tc_tiling_on_sc=False, needs_layout_passes=False),
        out_type=(jax.ShapeDtypeStruct((NC, SP, 64), jnp.float32),
                  jax.ShapeDtypeStruct((EP, 8), jnp.float32)),
        scratch_types=[pltpu.VMEM((2, ch), jnp.int32),
                       pltpu.VMEM((2, ch), jnp.int32),
                       pltpu.VMEM((2, ch, 80), jnp.float32),
                       pltpu.VMEM((2, ch, DS), jnp.float32),
                       pltpu.VMEM((ch, DS), jnp.float32),
                       pltpu.VMEM((ch, 64), jnp.float32),
                       pltpu.VMEM((ch, 8), jnp.float32),
                       pltpu.VMEM_SHARED((SP, 64), jnp.float32),
                       pltpu.SemaphoreType.DMA,
                       pltpu.SemaphoreType.DMA],
    )
    def k(tc_h, dd_h, w_h, di_h, si_h, z64_h, z8_h, agg_h, dz_h,
          dvi2, svi2, gv2, gdd2, wbuf, mbuf, dzbuf, acc64,
          sg0, sg1):
        cid = lax.axis_index("c")
        sid = lax.axis_index("s")
        r0 = sid * stripe
        pltpu.sync_copy(z64_h.at[pl.ds(r0, stripe)], acc64.at[pl.ds(r0, stripe)])
        pltpu.sync_copy(z8_h.at[pl.ds(0, ch)], dzbuf)
        plsc.subcore_barrier()
        lanes = lax.iota(jnp.int32, 16)
        SG = (sg0, sg1)

        def issue(j, pp):
            b = sid * per_t + j * ch
            pltpu.sync_copy(di_h.at[pl.ds(b, ch)], dvi2.at[pp])
            pltpu.sync_copy(si_h.at[cid, pl.ds(b, ch)], svi2.at[pp])
            pltpu.async_copy(tc_h.at[svi2.at[pp]], gv2.at[pp], SG[pp])
            pltpu.async_copy(dd_h.at[dvi2.at[pp]], gdd2.at[pp], SG[pp])

        def waitb(pp):
            pltpu.make_async_copy(tc_h.at[svi2.at[pp]], gv2.at[pp],
                                  SG[pp]).wait()
            pltpu.make_async_copy(dd_h.at[dvi2.at[pp]], gdd2.at[pp],
                                  SG[pp]).wait()

        issue(0, 0)

        def chunk(jj, carry):
          for pp in (0, 1):
            i = 2 * jj + pp
            b = sid * per_t + i * ch
            nxt = i + 1
            nxt = nxt - (nxt // iters) * iters
            issue(nxt, pp ^ 1)
            waitb(pp)
            pltpu.sync_copy(w_h.at[pl.ds(b, ch), :], wbuf)
            gv = gv2.at[pp]
            gdd = gdd2.at[pp]

            def group(g, c3):
                rows = g * 16 + lanes

                def col(ref, c):
                    return plsc.load_gather(
                        ref, [rows, jnp.full((16,), c, jnp.int32)])

                a = []
                for h in range(NHEAD):
                    a.append(col(wbuf, h) / col(gdd, h))
                is0 = (cid == 0)
                a_lo = jnp.where(is0, a[0], a[2])
                a_hi = jnp.where(is0, a[1], a[3])
                for c in range(64):
                    av = a_lo if c < 32 else a_hi
                    m = av * col(gv, c)
                    plsc.store_scatter(
                        mbuf, [rows, jnp.full((16,), c, jnp.int32)], m)
                csc = jnp.zeros((16,), jnp.float32)
                for h in range(NHEAD):
                    csc = csc + a[h] * col(gv, 64 + h)
                for t in range(3):
                    dz = (col(gdd, NHEAD + t) - col(gv, 68 + t)) * csc
                    plsc.store_scatter(
                        dzbuf, [rows, jnp.full((16,), t, jnp.int32)], dz)
                return c3

            lax.fori_loop(0, ch // 16, group, 0)
            pltpu.sync_copy(mbuf, acc64.at[dvi2.at[pp]], add=True)

            @pl.when(cid == 0)
            def _():
                pltpu.sync_copy(dzbuf, dz_h.at[pl.ds(b, ch)])

          return carry

        lax.fori_loop(0, iters // 2, chunk, 0)
        waitb(0)
        plsc.subcore_barrier()
        pltpu.sync_copy(acc64.at[pl.ds(r0, stripe)],
                        agg_h.at[cid, pl.ds(r0, stripe), :])

    return k(tcat, ddst, w16, dst, src2, z64, z8)


def _tc(body, grid, in_specs, out_shapes, out_specs):
    return pl.pallas_call(
        body, grid=grid, in_specs=in_specs,
        out_shape=out_shapes, out_specs=out_specs)


def _full(shape):
    return pl.BlockSpec(shape, lambda i: tuple(0 for _ in shape))


def _rows(shape):
    nd = len(shape)
    return pl.BlockSpec(shape, lambda i: (i,) + (0,) * (nd - 1))


def _mid(shape):
    return pl.BlockSpec(shape, lambda i: (0, i, 0))


SB = 784
RB = 2048


def _tables(hb, zc, wq, wk, wv, wet, wc, tdst_ref, tsrca_ref, tsrcc_ref):
    q = jnp.dot(hb, wq, preferred_element_type=jnp.float32)
    kk = jnp.dot(hb, wk, preferred_element_type=jnp.float32)
    v = jnp.dot(hb, wv, preferred_element_type=jnp.float32)
    a = jnp.concatenate(
        [jnp.dot(q[:, 32 * h:32 * h + 32], wet[32 * h:32 * h + 32, :],
                 preferred_element_type=jnp.float32) for h in range(NHEAD)],
        axis=1)
    vc = jnp.sum((v * wc[:, 0][None, :]).reshape(-1, NHEAD, DH), axis=-1)
    z3 = zc[:, 0:3]
    zpad = jnp.zeros((hb.shape[0], D_DST - 259), jnp.float32)
    tdst_ref[...] = jnp.concatenate([q, a, z3, zpad], axis=1)
    spad = jnp.zeros((hb.shape[0], D_SRC - 131), jnp.float32)
    tsrca_ref[...] = jnp.concatenate([kk, z3, spad], axis=1)
    cpad = jnp.zeros((hb.shape[0], 80 - 71), jnp.float32)
    h0 = jnp.concatenate([v[:, 0:64], vc[:, 0:2], vc[:, 2:4], z3, cpad], axis=1)
    h1 = jnp.concatenate([v[:, 64:128], vc[:, 0:2], vc[:, 2:4], z3, cpad], axis=1)
    tsrcc_ref[...] = jnp.stack([h0, h1], axis=0)


def _k_pool0_body(hs_ref, zcnt_ref, degp_ref, wq_ref, wk_ref, wv_ref,
                  wet_ref, wc_ref, hb_ref, zc_ref, degv_ref,
                  tdst_ref, tsrca_ref, tsrcc_ref):
    zc01 = zcnt_ref[0] + zcnt_ref[1]
    cnt = jnp.maximum(zc01[:, 3:4], 1.0)
    hb = jnp.concatenate([hs_ref[0], hs_ref[1]], axis=1) / cnt
    z3 = zc01[:, 0:3] / cnt
    deg01 = degp_ref[0] + degp_ref[1]
    deg = jnp.maximum(deg01[:, 0:1], 1.0)
    pad13 = jnp.zeros((SB, DS - 3), jnp.float32)
    zc = jnp.concatenate([z3, pad13], axis=1)
    hb_ref[...] = hb
    zc_ref[...] = zc
    pad15 = jnp.zeros((SB, DS - 1), jnp.float32)
    degv_ref[...] = jnp.concatenate([deg, pad15], axis=1)
    _tables(hb, zc, wq_ref[...], wk_ref[...], wv_ref[...], wet_ref[...],
            wc_ref[...], tdst_ref, tsrca_ref, tsrcc_ref)


def _k_upd_body(hc_ref, zc_ref, agg_ref, dzp_ref, degv_ref, wo_ref,
                wq_ref, wk_ref, wv_ref, wet_ref, wc_ref,
                hcn_ref, zcn_ref, tdst_ref, tsrca_ref, tsrcc_ref):
    agg = jnp.concatenate([agg_ref[0], agg_ref[1]], axis=1)
    hc = hc_ref[...] + jnp.dot(agg, wo_ref[...],
                               preferred_element_type=jnp.float32)
    dz01 = dzp_ref[0] + dzp_ref[1]
    deg = degv_ref[...][:, 0:1]
    z3 = zc_ref[...][:, 0:3] + dz01[:, 0:3] / deg
    pad13 = jnp.zeros((SB, DS - 3), jnp.float32)
    zc = jnp.concatenate([z3, pad13], axis=1)
    hcn_ref[...] = hc
    zcn_ref[...] = zc
    _tables(hc, zc, wq_ref[...], wk_ref[...], wv_ref[...], wet_ref[...],
            wc_ref[...], tdst_ref, tsrca_ref, tsrcc_ref)


_RBF_DEN = 2.0 * (CUTOFF / NRBF) ** 2
_ISQ = 1.0 / np.sqrt(DH)


def _k_logits_body(gd_ref, gs_ref, ea_ref, lo_ref):
    i = pl.program_id(0)
    gd = gd_ref[...]
    gs = gs_ref[...]
    rel = gd[:, 256:259] - gs[:, 128:131]
    d = jnp.sqrt(jnp.sum(rel * rel, axis=1, keepdims=True) + 1e-8)
    centers = (lax.broadcasted_iota(jnp.int32, (1, NRBF), 1).astype(jnp.float32)
               * (CUTOFF / (NRBF - 1)))
    rbf = jnp.exp(-((d - centers) ** 2) / _RBF_DEN)
    feat = jnp.concatenate([rbf, ea_ref[...]], axis=1)
    qk = jnp.sum((gd[:, 0:128] * gs[:, 0:128]).reshape(RB, NHEAD, DH), axis=-1)
    fa = jnp.sum(gd[:, 128:256].reshape(RB, NHEAD, DH) * feat[:, None, :],
                 axis=-1)
    w = jnp.exp((qk + fa) * _ISQ)
    rows = i * RB + lax.broadcasted_iota(jnp.int32, (RB, 1), 0)
    w = jnp.where(rows < E, w, 0.0)
    lo_ref[...] = jnp.concatenate(
        [w, jnp.zeros((RB, DS - NHEAD), jnp.float32)], axis=1)


def _k_ctab_body(lsum_ref, degv_ref, ct_ref):
    lsum = lsum_ref[0] + lsum_ref[1]
    c = lsum[:, 0:NHEAD] / degv_ref[...][:, 0:1]
    ct_ref[...] = jnp.concatenate(
        [c, jnp.zeros((SB, DS - NHEAD), jnp.float32)], axis=1)


def _k_w_body(lo_ref, gc_ref, w_ref):
    i = pl.program_id(0)
    w = jnp.exp(lo_ref[...][:, 0:NHEAD] - gc_ref[...][:, 0:NHEAD])
    rows = i * RB + lax.broadcasted_iota(jnp.int32, (RB, 1), 0)
    w = jnp.where(rows < E, w, 0.0)
    w_ref[...] = jnp.concatenate(
        [w, jnp.zeros((RB, DS - NHEAD), jnp.float32)], axis=1)


def _k_ddst_body(denp_ref, zc_ref, dd_ref):
    den = denp_ref[0] + denp_ref[1]
    dd = den[:, 0:NHEAD] + 1e-8
    z3 = zc_ref[...][:, 0:3]
    dd_ref[...] = jnp.concatenate(
        [dd, z3, jnp.zeros((SB, DS - NHEAD - 3), jnp.float32)], axis=1)


def _k_msg_body(w_ref, gdd_ref, gs_ref, msg_ref, dzv_ref):
    gdd = gdd_ref[...]
    gs = gs_ref[...]
    alpha = w_ref[...][:, 0:NHEAD] / gdd[:, 0:NHEAD]
    v = gs[:, 0:128].reshape(RB, NHEAD, DH)
    msg = (alpha[:, :, None] * v).reshape(RB, HIDDEN)
    msg_ref[...] = jnp.stack([msg[:, 0:64], msg[:, 64:128]], axis=0)
    csc = jnp.sum(alpha * gs[:, 128:132], axis=1, keepdims=True)
    rel = gdd[:, NHEAD:NHEAD + 3] - gs[:, 132:135]
    dzv_ref[...] = jnp.concatenate(
        [rel * csc, jnp.zeros((RB, DS - 3), jnp.float32)], axis=1)


def _k_final_body(hc_ref, agg_ref, wo_ref, br_ref, br3_ref):
    agg = jnp.concatenate([agg_ref[0], agg_ref[1]], axis=1)
    hc = hc_ref[...] + jnp.dot(agg, wo_ref[...],
                               preferred_element_type=jnp.float32)
    nrm = jnp.sqrt(jnp.sum(hc * hc, axis=1, keepdims=True))
    br = hc / (nrm + 1e-12)
    br_ref[...] = br
    br3_ref[...] = jnp.stack([br[:, 0:64], br[:, 64:128]], axis=0)


def _k_graph_body(grp_ref, gr_ref):
    g = jnp.concatenate([grp_ref[0], grp_ref[1]], axis=1)
    nrm = jnp.sqrt(jnp.sum(g * g, axis=1, keepdims=True))
    gr_ref[...] = g / (nrm + 1e-12)


def kernel(H, Z, block_id, batch_id, edges, edge_attr, Wq, Wk, Wv, We, Wo, Wc):
    f32 = jnp.float32
    src = jnp.pad(edges[0].astype(jnp.int32), (0, EP - E))
    dst = jnp.pad(edges[1].astype(jnp.int32), (0, EP - E))
    ea_p = jnp.pad(edge_attr, ((0, EP - E), (0, 0)))
    emask = (jnp.arange(EP) < E).astype(f32)[:, None]
    degvals = jnp.pad(emask, ((0, 0), (0, DS - 1)))
    bid_p = jnp.pad(block_id.astype(jnp.int32), (0, NUP - N_UNIT))
    hp = jnp.pad(H, ((0, NUP - N_UNIT), (0, 0)))
    h3 = jnp.stack([hp[:, 0:64], hp[:, 64:128]], axis=0)
    zuv = jnp.pad(
        jnp.concatenate([Z.reshape(N_UNIT, 3), jnp.ones((N_UNIT, 1), f32)],
                        axis=1),
        ((0, NUP - N_UNIT), (0, DS - 4)))
    batch_p = jnp.pad(batch_id.astype(jnp.int32), (0, SP - N_BLOCK))
    z64 = jnp.zeros((SP, 64), f32)
    z64g = jnp.zeros((SGP, 64), f32)
    z16 = jnp.zeros((SP, DS), f32)
    z8 = jnp.zeros((SP, 8), f32)
    src2 = jnp.stack([src, src + SP], axis=0)
    wet = jnp.transpose(We, (0, 2, 1))

    hsum3 = _sc_scatter_cols(h3, bid_p, SP, z64)
    zcnt = _sc_scatter_small(zuv, bid_p, SP, z16)
    degp = _sc_scatter_small(degvals, dst, SP, z16)

    gs = (32,)
    w_spec = _full((HIDDEN, HIDDEN))
    wet_spec = _full((HIDDEN, 32))
    wc_spec = _full((HIDDEN, 1))
    tbl_out = (jax.ShapeDtypeStruct((SP, D_DST), f32),
               jax.ShapeDtypeStruct((SP, D_SRC), f32),
               jax.ShapeDtypeStruct((NC, SP, 80), f32))
    tbl_spec = (_rows((SB, D_DST)), _rows((SB, D_SRC)), _mid((NC, SB, 80)))

    hb, zc, degv, tdst, tsrca, tsrcc = _tc(
        _k_pool0_body, gs,
        [_mid((NC, SB, 64)), _mid((NC, SB, DS)), _mid((NC, SB, DS)),
         w_spec, w_spec, w_spec, wet_spec, wc_spec],
        (jax.ShapeDtypeStruct((SP, HIDDEN), f32),
         jax.ShapeDtypeStruct((SP, DS), f32),
         jax.ShapeDtypeStruct((SP, DS), f32)) + tbl_out,
        (_rows((SB, HIDDEN)), _rows((SB, DS)), _rows((SB, DS))) + tbl_spec,
    )(hsum3, zcnt, degp, Wq[0], Wk[0], Wv[0], wet[0], Wc[0])

    ge = (EP // RB,)
    hc = hb
    for l in range(LAYERS):
        if l > 0:
            hc, zc, tdst, tsrca, tsrcc = _tc(
                _k_upd_body, gs,
                [_rows((SB, HIDDEN)), _rows((SB, DS)), _mid((NC, SB, 64)),
                 _mid((NC, SB, 8)), _rows((SB, DS)),
                 w_spec, w_spec, w_spec, w_spec, wet_spec, wc_spec],
                (jax.ShapeDtypeStruct((SP, HIDDEN), f32),
                 jax.ShapeDtypeStruct((SP, DS), f32)) + tbl_out,
                (_rows((SB, HIDDEN)), _rows((SB, DS))) + tbl_spec,
            )(hc, zc, agg3, dzp, degv, Wo[l - 1],
              Wq[l], Wk[l], Wv[l], wet[l], Wc[l])

        w16, denp = _sc_pass_a(tdst, tsrca, dst, src, ea_p, z16)
        ddst = _tc(
            _k_ddst_body, gs, [_mid((NC, SB, DS)), _rows((SB, DS))],
            jax.ShapeDtypeStruct((SP, DS), f32), _rows((SB, DS)),
        )(denp, zc)
        tcat = tsrcc.reshape(NC * SP, 80)
        agg3, dzv = _sc_pass_c(tcat, ddst, w16, dst, src2, z64, z8)
        dzp = _sc_scatter_small(dzv, dst, SP, z8, w=8)

    br, br3 = _tc(
        _k_final_body, gs,
        [_rows((SB, HIDDEN)), _mid((NC, SB, 64)), w_spec],
        (jax.ShapeDtypeStruct((SP, HIDDEN), f32),
         jax.ShapeDtypeStruct((NC, SP, 64), f32)),
        (_rows((SB, HIDDEN)), _mid((NC, SB, 64))),
    )(hc, agg3, Wo[LAYERS - 1])

    grp = _sc_scatter_cols(br3, batch_p, SGP, z64g, ch=112)
    gr = _tc(
        _k_graph_body, (1,), [_full((NC, SGP, 64))],
        jax.ShapeDtypeStruct((SGP, HIDDEN), f32), _full((SGP, HIDDEN)),
    )(grp)

    return (hb[:N_BLOCK], br[:N_BLOCK], gr[:N_GRAPH])


# speedup vs baseline: 2.3279x; 1.1472x over previous
import functools

import jax
import jax.numpy as jnp
import numpy as np
from jax import lax
from jax.experimental import pallas as pl
from jax.experimental.pallas import tpu as pltpu
from jax.experimental.pallas import tpu_sc as plsc

N_UNIT = 100000
N_BLOCK = 25000
N_GRAPH = 64
E = 400000
HIDDEN = 128
NRBF = 16
EDGE = 16
NHEAD = 4
DH = HIDDEN // NHEAD
LAYERS = 3
CUTOFF = 7.0

NC, NS = 2, 16
NW = NC * NS
SP = 25088
EP = 409600
NUP = 102400
SGP = 128
DS = 16
D_DST = 272
D_SRC = 144

_mesh = lambda: plsc.VectorSubcoreMesh(core_axis_name="c", subcore_axis_name="s")


def _sc_gather2(tab1, idx1, tab2, idx2, d1, d2, ch=128):
    n = idx1.shape[0]
    per_w = n // NW
    iters = per_w // ch

    @functools.partial(
        pl.kernel, mesh=_mesh(),
        compiler_params=pltpu.CompilerParams(
            use_tc_tiling_on_sc=False, needs_layout_passes=False),
        out_type=(jax.ShapeDtypeStruct((n, d1), jnp.float32),
                  jax.ShapeDtypeStruct((n, d2), jnp.float32)),
        scratch_types=[pltpu.VMEM((ch,), jnp.int32),
                       pltpu.VMEM((ch,), jnp.int32),
                       pltpu.VMEM((ch, d1), jnp.float32),
                       pltpu.VMEM((ch, d2), jnp.float32),
                       pltpu.SemaphoreType.DMA,
                       pltpu.SemaphoreType.DMA],
    )
    def k(t1, i1, t2, i2, o1, o2, iv1, iv2, r1, r2, s1, s2):
        wid = lax.axis_index("s") * NC + lax.axis_index("c")
        base = wid * per_w

        def body(i, carry):
            b = base + i * ch
            pltpu.sync_copy(i1.at[pl.ds(b, ch)], iv1)
            pltpu.sync_copy(i2.at[pl.ds(b, ch)], iv2)
            c1 = pltpu.async_copy(t1.at[iv1], r1, s1)
            c2 = pltpu.async_copy(t2.at[iv2], r2, s2)
            c1.wait()
            c2.wait()
            pltpu.sync_copy(r1, o1.at[pl.ds(b, ch)])
            pltpu.sync_copy(r2, o2.at[pl.ds(b, ch)])
            return carry

        lax.fori_loop(0, iters, body, 0)

    return k(tab1, idx1, tab2, idx2)


def _sc_gather1(tab, idx, d, ch=128):
    n = idx.shape[0]
    per_w = n // NW
    iters = per_w // ch

    @functools.partial(
        pl.kernel, mesh=_mesh(),
        compiler_params=pltpu.CompilerParams(
            use_tc_tiling_on_sc=False, needs_layout_passes=False),
        out_type=jax.ShapeDtypeStruct((n, d), jnp.float32),
        scratch_types=[pltpu.VMEM((ch,), jnp.int32),
                       pltpu.VMEM((ch, d), jnp.float32),
                       pltpu.SemaphoreType.DMA],
    )
    def k(t1, i1, o1, iv1, r1, s1):
        wid = lax.axis_index("s") * NC + lax.axis_index("c")
        base = wid * per_w

        def body(i, carry):
            b = base + i * ch
            pltpu.sync_copy(i1.at[pl.ds(b, ch)], iv1)
            pltpu.async_copy(t1.at[iv1], r1, s1).wait()
            pltpu.sync_copy(r1, o1.at[pl.ds(b, ch)])
            return carry

        lax.fori_loop(0, iters, body, 0)

    return k(tab, idx)


def _sc_scatter_cols(vals3, idx, s_out, zeros64, ch=128):
    n = idx.shape[0]
    per_t = n // NS
    iters = per_t // ch
    stripe = s_out // NS

    @functools.partial(
        pl.kernel, mesh=_mesh(),
        compiler_params=pltpu.CompilerParams(
            use_tc_tiling_on_sc=False, needs_layout_passes=False),
        out_type=jax.ShapeDtypeStruct((NC, s_out, 64), jnp.float32),
        scratch_types=[pltpu.VMEM((ch,), jnp.int32),
                       pltpu.VMEM((ch, 64), jnp.float32),
                       pltpu.VMEM_SHARED((s_out, 64), jnp.float32)],
    )
    def k(v_hbm, i_hbm, z_hbm, o_hbm, iv, vv, acc):
        cid = lax.axis_index("c")
        sid = lax.axis_index("s")
        r0 = sid * stripe
        pltpu.sync_copy(z_hbm.at[pl.ds(r0, stripe)], acc.at[pl.ds(r0, stripe)])
        plsc.subcore_barrier()

        def body(i, carry):
            b = sid * per_t + i * ch
            pltpu.sync_copy(i_hbm.at[pl.ds(b, ch)], iv)
            pltpu.sync_copy(v_hbm.at[cid, pl.ds(b, ch), :], vv)
            pltpu.sync_copy(vv, acc.at[iv], add=True)
            return carry

        lax.fori_loop(0, iters, body, 0)
        plsc.subcore_barrier()
        pltpu.sync_copy(acc.at[pl.ds(r0, stripe)],
                        o_hbm.at[cid, pl.ds(r0, stripe), :])

    return k(vals3, idx, zeros64)


def _sc_scatter_small(vals, idx, s_out, zeros16, ch=128, w=DS):
    n = idx.shape[0]
    per_w = n // NW
    iters = per_w // ch
    stripe = s_out // NS

    @functools.partial(
        pl.kernel, mesh=_mesh(),
        compiler_params=pltpu.CompilerParams(
            use_tc_tiling_on_sc=False, needs_layout_passes=False),
        out_type=jax.ShapeDtypeStruct((NC, s_out, w), jnp.float32),
        scratch_types=[pltpu.VMEM((ch,), jnp.int32),
                       pltpu.VMEM((ch, w), jnp.float32),
                       pltpu.VMEM_SHARED((s_out, w), jnp.float32)],
    )
    def k(v_hbm, i_hbm, z_hbm, o_hbm, iv, vv, acc):
        cid = lax.axis_index("c")
        sid = lax.axis_index("s")
        wid = sid * NC + cid
        r0 = sid * stripe
        pltpu.sync_copy(z_hbm.at[pl.ds(r0, stripe)], acc.at[pl.ds(r0, stripe)])
        plsc.subcore_barrier()

        def body(i, carry):
            b = wid * per_w + i * ch
            pltpu.sync_copy(i_hbm.at[pl.ds(b, ch)], iv)
            pltpu.sync_copy(v_hbm.at[pl.ds(b, ch), :], vv)
            pltpu.sync_copy(vv, acc.at[iv], add=True)
            return carry

        lax.fori_loop(0, iters, body, 0)
        plsc.subcore_barrier()
        pltpu.sync_copy(acc.at[pl.ds(r0, stripe)],
                        o_hbm.at[cid, pl.ds(r0, stripe), :])

    return k(vals, idx, zeros16)


def _vsqrt(r2):
    ib = plsc.bitcast(r2, jnp.int32)
    x = plsc.bitcast((ib >> 1) + jnp.int32(0x1FBD1DF5), jnp.float32)
    for _ in range(3):
        x = 0.5 * (x + r2 / x)
    return x


def _sc_pass_a(tdst, tsrca, dst, src, ea, z16, ch=80):
    per_w = EP // NW
    iters = per_w // ch
    stripe = SP // NS
    isq = float(1.0 / np.sqrt(DH))
    inv2s = float(1.0 / _RBF_DEN)
    cstep = float(CUTOFF / (NRBF - 1))

    @functools.partial(
        pl.kernel, mesh=_mesh(),
        compiler_params=pltpu.CompilerParams(
            use_tc_tiling_on_sc=False, needs_layout_passes=False),
        out_type=(jax.ShapeDtypeStruct((EP, DS), jnp.float32),
                  jax.ShapeDtypeStruct((NC, SP, DS), jnp.float32)),
        scratch_types=[pltpu.VMEM((2, ch), jnp.int32),
                       pltpu.VMEM((2, ch), jnp.int32),
                       pltpu.VMEM((2, ch, D_DST), jnp.float32),
                       pltpu.VMEM((2, ch, D_SRC), jnp.float32),
                       pltpu.VMEM((ch, EDGE), jnp.float32),
                       pltpu.VMEM((ch, DS), jnp.float32),
                       pltpu.VMEM_SHARED((SP, DS), jnp.float32),
                       pltpu.SemaphoreType.DMA,
                       pltpu.SemaphoreType.DMA],
    )
    def k(td_h, ts_h, di_h, si_h, ea_h, z_h, w_h, dp_h,
          dvi2, svi2, gd2, gs2, eab, wb, dacc, s1, s2):
        cid = lax.axis_index("c")
        sid = lax.axis_index("s")
        wid = sid * NC + cid
        r0 = sid * stripe
        pltpu.sync_copy(z_h.at[pl.ds(r0, stripe)], dacc.at[pl.ds(r0, stripe)])
        pltpu.sync_copy(z_h.at[pl.ds(0, ch)], wb)
        plsc.subcore_barrier()
        lanes = lax.iota(jnp.int32, 16)
        SG = (s1, s2)

        def issue(j, pp):
            b = wid * per_w + j * ch
            pltpu.sync_copy(di_h.at[pl.ds(b, ch)], dvi2.at[pp])
            pltpu.sync_copy(si_h.at[pl.ds(b, ch)], svi2.at[pp])
            pltpu.async_copy(td_h.at[dvi2.at[pp]], gd2.at[pp], SG[pp])
            pltpu.async_copy(ts_h.at[svi2.at[pp]], gs2.at[pp], SG[pp])

        def waitb(pp):
            pltpu.make_async_copy(td_h.at[dvi2.at[pp]], gd2.at[pp],
                                  SG[pp]).wait()
            pltpu.make_async_copy(ts_h.at[svi2.at[pp]], gs2.at[pp],
                                  SG[pp]).wait()

        issue(0, 0)

        def chunk(jj, carry):
          for pp in (0, 1):
            i = 2 * jj + pp
            b = wid * per_w + i * ch
            nxt = i + 1
            nxt = nxt - (nxt // iters) * iters
            issue(nxt, pp ^ 1)
            waitb(pp)
            pltpu.sync_copy(ea_h.at[pl.ds(b, ch), :], eab)
            gd = gd2.at[pp]
            gs = gs2.at[pp]

            def group(g, c3):
                rows = g * 16 + lanes

                def col(ref, c):
                    return plsc.load_gather(
                        ref, [rows, jnp.full((16,), c, jnp.int32)])

                rel2 = jnp.full((16,), 1e-8, jnp.float32)
                for t in range(3):
                    dt = col(gd, 256 + t) - col(gs, 128 + t)
                    rel2 = rel2 + dt * dt
                d = _vsqrt(rel2)
                feat = []
                for j in range(NRBF):
                    t = d - (cstep * j)
                    feat.append(jnp.exp(-(t * t) * inv2s))
                for j in range(EDGE):
                    feat.append(col(eab, j))
                valid = (b + rows) < E
                for h in range(NHEAD):
                    acc = jnp.zeros((16,), jnp.float32)
                    for c in range(DH):
                        acc = acc + col(gd, 32 * h + c) * col(gs, 32 * h + c)
                    for j in range(32):
                        acc = acc + feat[j] * col(gd, 128 + 32 * h + j)
                    w = jnp.exp(acc * isq)
                    w = jnp.where(valid, w, 0.0)
                    plsc.store_scatter(
                        wb, [rows, jnp.full((16,), h, jnp.int32)], w)
                return c3

            lax.fori_loop(0, ch // 16, group, 0)
            pltpu.sync_copy(wb, w_h.at[pl.ds(b, ch)])
            pltpu.sync_copy(wb, dacc.at[dvi2.at[pp]], add=True)
          return carry

        lax.fori_loop(0, iters // 2, chunk, 0)
        waitb(0)
        plsc.subcore_barrier()
        pltpu.sync_copy(dacc.at[pl.ds(r0, stripe)],
                        dp_h.at[cid, pl.ds(r0, stripe), :])

    return k(tdst, tsrca, dst, src, ea, z16)


def _sc_pass_c(tcat, ddst, w16, dst, src2, z64, z8, ch=80):
    per_t = EP // NS
    iters = per_t // ch
    stripe = SP // NS

    @functools.partial(
        pl.kernel, mesh=_mesh(),
        compiler_params=pltpu.CompilerParams(
            use_tc_tiling_on_sc=False, needs_layout_passes=False),
        out_type=(jax.ShapeDtypeStruct((NC, SP, 64), jnp.float32),
                  jax.ShapeDtypeStruct((EP, 8), jnp.float32)),
        scratch_types=[pltpu.VMEM((2, ch), jnp.int32),
                       pltpu.VMEM((2, ch), jnp.int32),
                       pltpu.VMEM((2, ch, 80), jnp.float32),
                       pltpu.VMEM((2, ch, DS), jnp.float32),
                       pltpu.VMEM((ch, DS), jnp.float32),
                       pltpu.VMEM((ch, 64), jnp.float32),
                       pltpu.VMEM((ch, 8), jnp.float32),
                       pltpu.VMEM_SHARED((SP, 64), jnp.float32),
                       pltpu.SemaphoreType.DMA,
                       pltpu.SemaphoreType.DMA],
    )
    def k(tc_h, dd_h, w_h, di_h, si_h, z64_h, z8_h, agg_h, dz_h,
          dvi2, svi2, gv2, gdd2, wbuf, mbuf, dzbuf, acc64,
          sg0, sg1):
        cid = lax.axis_index("c")
        sid = lax.axis_index("s")
        r0 = sid * stripe
        pltpu.sync_copy(z64_h.at[pl.ds(r0, stripe)], acc64.at[pl.ds(r0, stripe)])
        pltpu.sync_copy(z8_h.at[pl.ds(0, ch)], dzbuf)
        plsc.subcore_barrier()
        lanes = lax.iota(jnp.int32, 16)
        SG = (sg0, sg1)

        def issue(j, pp):
            b = sid * per_t + j * ch
            pltpu.sync_copy(di_h.at[pl.ds(b, ch)], dvi2.at[pp])
            pltpu.sync_copy(si_h.at[cid, pl.ds(b, ch)], svi2.at[pp])
            pltpu.async_copy(tc_h.at[svi2.at[pp]], gv2.at[pp], SG[pp])
            pltpu.async_copy(dd_h.at[dvi2.at[pp]], gdd2.at[pp], SG[pp])

        def waitb(pp):
            pltpu.make_async_copy(tc_h.at[svi2.at[pp]], gv2.at[pp],
                                  SG[pp]).wait()
            pltpu.make_async_copy(dd_h.at[dvi2.at[pp]], gdd2.at[pp],
                                  SG[pp]).wait()

        issue(0, 0)

        def chunk(jj, carry):
          for pp in (0, 1):
            i = 2 * jj + pp
            b = sid * per_t + i * ch
            nxt = i + 1
            nxt = nxt - (nxt // iters) * iters
            issue(nxt, pp ^ 1)
            waitb(pp)
            pltpu.sync_copy(w_h.at[pl.ds(b, ch), :], wbuf)
            gv = gv2.at[pp]
            gdd = gdd2.at[pp]

            def group(g, c3):
                rows = g * 16 + lanes

                def col(ref, c):
                    return plsc.load_gather(
                        ref, [rows, jnp.full((16,), c, jnp.int32)])

                a = []
                for h in range(NHEAD):
                    a.append(col(wbuf, h) / col(gdd, h))
                is0 = (cid == 0)
                a_lo = jnp.where(is0, a[0], a[2])
                a_hi = jnp.where(is0, a[1], a[3])
                for c in range(64):
                    av = a_lo if c < 32 else a_hi
                    m = av * col(gv, c)
                    plsc.store_scatter(
                        mbuf, [rows, jnp.full((16,), c, jnp.int32)], m)
                csc = jnp.zeros((16,), jnp.float32)
                for h in range(NHEAD):
                    csc = csc + a[h] * col(gv, 64 + h)
                for t in range(3):
                    dz = (col(gdd, NHEAD + t) - col(gv, 68 + t)) * csc
                    plsc.store_scatter(
                        dzbuf, [rows, jnp.full((16,), t, jnp.int32)], dz)
                return c3

            lax.fori_loop(0, ch // 16, group, 0)
            pltpu.sync_copy(mbuf, acc64.at[dvi2.at[pp]], add=True)

            @pl.when(cid == 0)
            def _():
                pltpu.sync_copy(dzbuf, dz_h.at[pl.ds(b, ch)])

          return carry

        lax.fori_loop(0, iters // 2, chunk, 0)
        waitb(0)
        plsc.subcore_barrier()
        pltpu.sync_copy(acc64.at[pl.ds(r0, stripe)],
                        agg_h.at[cid, pl.ds(r0, stripe), :])

    return k(tcat, ddst, w16, dst, src2, z64, z8)


def _tc(body, grid, in_specs, out_shapes, out_specs):
    return pl.pallas_call(
        body, grid=grid, in_specs=in_specs,
        out_shape=out_shapes, out_specs=out_specs)


def _full(shape):
    return pl.BlockSpec(shape, lambda i: tuple(0 for _ in shape))


def _rows(shape):
    nd = len(shape)
    return pl.BlockSpec(shape, lambda i: (i,) + (0,) * (nd - 1))


def _mid(shape):
    return pl.BlockSpec(shape, lambda i: (0, i, 0))


SB = 784
RB = 2048


def _tables(hb, zc, wq, wk, wv, wet, wc, tdst_ref, tsrca_ref, tsrcc_ref):
    q = jnp.dot(hb, wq, preferred_element_type=jnp.float32)
    kk = jnp.dot(hb, wk, preferred_element_type=jnp.float32)
    v = jnp.dot(hb, wv, preferred_element_type=jnp.float32)
    a = jnp.concatenate(
        [jnp.dot(q[:, 32 * h:32 * h + 32], wet[32 * h:32 * h + 32, :],
                 preferred_element_type=jnp.float32) for h in range(NHEAD)],
        axis=1)
    vc = jnp.sum((v * wc[:, 0][None, :]).reshape(-1, NHEAD, DH), axis=-1)
    z3 = zc[:, 0:3]
    zpad = jnp.zeros((hb.shape[0], D_DST - 259), jnp.float32)
    tdst_ref[...] = jnp.concatenate([q, a, z3, zpad], axis=1)
    spad = jnp.zeros((hb.shape[0], D_SRC - 131), jnp.float32)
    tsrca_ref[...] = jnp.concatenate([kk, z3, spad], axis=1)
    cpad = jnp.zeros((hb.shape[0], 80 - 71), jnp.float32)
    h0 = jnp.concatenate([v[:, 0:64], vc[:, 0:2], vc[:, 2:4], z3, cpad], axis=1)
    h1 = jnp.concatenate([v[:, 64:128], vc[:, 0:2], vc[:, 2:4], z3, cpad], axis=1)
    tsrcc_ref[...] = jnp.stack([h0, h1], axis=0)


def _k_pool0_body(hs_ref, zcnt_ref, degp_ref, wq_ref, wk_ref, wv_ref,
                  wet_ref, wc_ref, hb_ref, zc_ref, degv_ref,
                  tdst_ref, tsrca_ref, tsrcc_ref):
    zc01 = zcnt_ref[0] + zcnt_ref[1]
    cnt = jnp.maximum(zc01[:, 3:4], 1.0)
    hb = jnp.concatenate([hs_ref[0], hs_ref[1]], axis=1) / cnt
    z3 = zc01[:, 0:3] / cnt
    deg01 = degp_ref[0] + degp_ref[1]
    deg = jnp.maximum(deg01[:, 0:1], 1.0)
    pad13 = jnp.zeros((SB, DS - 3), jnp.float32)
    zc = jnp.concatenate([z3, pad13], axis=1)
    hb_ref[...] = hb
    zc_ref[...] = zc
    pad15 = jnp.zeros((SB, DS - 1), jnp.float32)
    degv_ref[...] = jnp.concatenate([deg, pad15], axis=1)
    _tables(hb, zc, wq_ref[...], wk_ref[...], wv_ref[...], wet_ref[...],
            wc_ref[...], tdst_ref, tsrca_ref, tsrcc_ref)


def _k_upd_body(hc_ref, zc_ref, agg_ref, dzp_ref, degv_ref, wo_ref,
                wq_ref, wk_ref, wv_ref, wet_ref, wc_ref,
                hcn_ref, zcn_ref, tdst_ref, tsrca_ref, tsrcc_ref):
    agg = jnp.concatenate([agg_ref[0], agg_ref[1]], axis=1)
    hc = hc_ref[...] + jnp.dot(agg, wo_ref[...],
                               preferred_element_type=jnp.float32)
    dz01 = dzp_ref[0] + dzp_ref[1]
    deg = degv_ref[...][:, 0:1]
    z3 = zc_ref[...][:, 0:3] + dz01[:, 0:3] / deg
    pad13 = jnp.zeros((SB, DS - 3), jnp.float32)
    zc = jnp.concatenate([z3, pad13], axis=1)
    hcn_ref[...] = hc
    zcn_ref[...] = zc
    _tables(hc, zc, wq_ref[...], wk_ref[...], wv_ref[...], wet_ref[...],
            wc_ref[...], tdst_ref, tsrca_ref, tsrcc_ref)


_RBF_DEN = 2.0 * (CUTOFF / NRBF) ** 2
_ISQ = 1.0 / np.sqrt(DH)


def _k_logits_body(gd_ref, gs_ref, ea_ref, lo_ref):
    i = pl.program_id(0)
    gd = gd_ref[...]
    gs = gs_ref[...]
    rel = gd[:, 256:259] - gs[:, 128:131]
    d = jnp.sqrt(jnp.sum(rel * rel, axis=1, keepdims=True) + 1e-8)
    centers = (lax.broadcasted_iota(jnp.int32, (1, NRBF), 1).astype(jnp.float32)
               * (CUTOFF / (NRBF - 1)))
    rbf = jnp.exp(-((d - centers) ** 2) / _RBF_DEN)
    feat = jnp.concatenate([rbf, ea_ref[...]], axis=1)
    qk = jnp.sum((gd[:, 0:128] * gs[:, 0:128]).reshape(RB, NHEAD, DH), axis=-1)
    fa = jnp.sum(gd[:, 128:256].reshape(RB, NHEAD, DH) * feat[:, None, :],
                 axis=-1)
    w = jnp.exp((qk + fa) * _ISQ)
    rows = i * RB + lax.broadcasted_iota(jnp.int32, (RB, 1), 0)
    w = jnp.where(rows < E, w, 0.0)
    lo_ref[...] = jnp.concatenate(
        [w, jnp.zeros((RB, DS - NHEAD), jnp.float32)], axis=1)


def _k_ctab_body(lsum_ref, degv_ref, ct_ref):
    lsum = lsum_ref[0] + lsum_ref[1]
    c = lsum[:, 0:NHEAD] / degv_ref[...][:, 0:1]
    ct_ref[...] = jnp.concatenate(
        [c, jnp.zeros((SB, DS - NHEAD), jnp.float32)], axis=1)


def _k_w_body(lo_ref, gc_ref, w_ref):
    i = pl.program_id(0)
    w = jnp.exp(lo_ref[...][:, 0:NHEAD] - gc_ref[...][:, 0:NHEAD])
    rows = i * RB + lax.broadcasted_iota(jnp.int32, (RB, 1), 0)
    w = jnp.where(rows < E, w, 0.0)
    w_ref[...] = jnp.concatenate(
        [w, jnp.zeros((RB, DS - NHEAD), jnp.float32)], axis=1)


def _k_ddst_body(denp_ref, zc_ref, dd_ref):
    den = denp_ref[0] + denp_ref[1]
    dd = den[:, 0:NHEAD] + 1e-8
    z3 = zc_ref[...][:, 0:3]
    dd_ref[...] = jnp.concatenate(
        [dd, z3, jnp.zeros((SB, DS - NHEAD - 3), jnp.float32)], axis=1)


def _k_msg_body(w_ref, gdd_ref, gs_ref, msg_ref, dzv_ref):
    gdd = gdd_ref[...]
    gs = gs_ref[...]
    alpha = w_ref[...][:, 0:NHEAD] / gdd[:, 0:NHEAD]
    v = gs[:, 0:128].reshape(RB, NHEAD, DH)
    msg = (alpha[:, :, None] * v).reshape(RB, HIDDEN)
    msg_ref[...] = jnp.stack([msg[:, 0:64], msg[:, 64:128]], axis=0)
    csc = jnp.sum(alpha * gs[:, 128:132], axis=1, keepdims=True)
    rel = gdd[:, NHEAD:NHEAD + 3] - gs[:, 132:135]
    dzv_ref[...] = jnp.concatenate(
        [rel * csc, jnp.zeros((RB, DS - 3), jnp.float32)], axis=1)


def _k_final_body(hc_ref, agg_ref, wo_ref, br_ref, br3_ref):
    agg = jnp.concatenate([agg_ref[0], agg_ref[1]], axis=1)
    hc = hc_ref[...] + jnp.dot(agg, wo_ref[...],
                               preferred_element_type=jnp.float32)
    nrm = jnp.sqrt(jnp.sum(hc * hc, axis=1, keepdims=True))
    br = hc / (nrm + 1e-12)
    br_ref[...] = br
    br3_ref[...] = jnp.stack([br[:, 0:64], br[:, 64:128]], axis=0)


def _k_graph_body(grp_ref, gr_ref):
    g = jnp.concatenate([grp_ref[0], grp_ref[1]], axis=1)
    nrm = jnp.sqrt(jnp.sum(g * g, axis=1, keepdims=True))
    gr_ref[...] = g / (nrm + 1e-12)


def kernel(H, Z, block_id, batch_id, edges, edge_attr, Wq, Wk, Wv, We, Wo, Wc):
    f32 = jnp.float32
    src = jnp.pad(edges[0].astype(jnp.int32), (0, EP - E))
    dst = jnp.pad(edges[1].astype(jnp.int32), (0, EP - E))
    ea_p = jnp.pad(edge_attr, ((0, EP - E), (0, 0)))
    emask = (jnp.arange(EP) < E).astype(f32)[:, None]
    degvals = jnp.pad(emask, ((0, 0), (0, DS - 1)))
    bid_p = jnp.pad(block_id.astype(jnp.int32), (0, NUP - N_UNIT))
    hp = jnp.pad(H, ((0, NUP - N_UNIT), (0, 0)))
    h3 = jnp.stack([hp[:, 0:64], hp[:, 64:128]], axis=0)
    zuv = jnp.pad(
        jnp.concatenate([Z.reshape(N_UNIT, 3), jnp.ones((N_UNIT, 1), f32)],
                        axis=1),
        ((0, NUP - N_UNIT), (0, DS - 4)))
    batch_p = jnp.pad(batch_id.astype(jnp.int32), (0, SP - N_BLOCK))
    z64 = jnp.zeros((SP, 64), f32)
    z64g = jnp.zeros((SGP, 64), f32)
    z16 = jnp.zeros((SP, DS), f32)
    z8 = jnp.zeros((SP, 8), f32)
    src2 = jnp.stack([src, src + SP], axis=0)
    wet = jnp.transpose(We, (0, 2, 1))

    hsum3 = _sc_scatter_cols(h3, bid_p, SP, z64)
    zcnt = _sc_scatter_small(zuv, bid_p, SP, z16)
    degp = _sc_scatter_small(degvals, dst, SP, z16)

    gs = (32,)
    w_spec = _full((HIDDEN, HIDDEN))
    wet_spec = _full((HIDDEN, 32))
    wc_spec = _full((HIDDEN, 1))
    tbl_out = (jax.ShapeDtypeStruct((SP, D_DST), f32),
               jax.ShapeDtypeStruct((SP, D_SRC), f32),
               jax.ShapeDtypeStruct((NC, SP, 80), f32))
    tbl_spec = (_rows((SB, D_DST)), _rows((SB, D_SRC)), _mid((NC, SB, 80)))

    hb, zc, degv, tdst, tsrca, tsrcc = _tc(
        _k_pool0_body, gs,
        [_mid((NC, SB, 64)), _mid((NC, SB, DS)), _mid((NC, SB, DS)),
         w_spec, w_spec, w_spec, wet_spec, wc_spec],
        (jax.ShapeDtypeStruct((SP, HIDDEN), f32),
         jax.ShapeDtypeStruct((SP, DS), f32),
         jax.ShapeDtypeStruct((SP, DS), f32)) + tbl_out,
        (_rows((SB, HIDDEN)), _rows((SB, DS)), _rows((SB, DS))) + tbl_spec,
    )(hsum3, zcnt, degp, Wq[0], Wk[0], Wv[0], wet[0], Wc[0])

    ge = (EP // RB,)
    hc = hb
    for l in range(LAYERS):
        if l > 0:
            hc, zc, tdst, tsrca, tsrcc = _tc(
                _k_upd_body, gs,
                [_rows((SB, HIDDEN)), _rows((SB, DS)), _mid((NC, SB, 64)),
                 _mid((NC, SB, 8)), _rows((SB, DS)),
                 w_spec, w_spec, w_spec, w_spec, wet_spec, wc_spec],
                (jax.ShapeDtypeStruct((SP, HIDDEN), f32),
                 jax.ShapeDtypeStruct((SP, DS), f32)) + tbl_out,
                (_rows((SB, HIDDEN)), _rows((SB, DS))) + tbl_spec,
            )(hc, zc, agg3, dzp, degv, Wo[l - 1],
              Wq[l], Wk[l], Wv[l], wet[l], Wc[l])

        w16, denp = _sc_pass_a(tdst, tsrca, dst, src, ea_p, z16)
        ddst = _tc(
            _k_ddst_body, gs, [_mid((NC, SB, DS)), _rows((SB, DS))],
            jax.ShapeDtypeStruct((SP, DS), f32), _rows((SB, DS)),
        )(denp, zc)
        tcat = tsrcc.reshape(NC * SP, 80)
        agg3, dzv = _sc_pass_c(tcat, ddst, w16, dst, src2, z64, z8)
        dzp = _sc_scatter_small(dzv, dst, SP, z8, w=8)

    br, br3 = _tc(
        _k_final_body, gs,
        [_rows((SB, HIDDEN)), _mid((NC, SB, 64)), w_spec],
        (jax.ShapeDtypeStruct((SP, HIDDEN), f32),
         jax.ShapeDtypeStruct((NC, SP, 64), f32)),
        (_rows((SB, HIDDEN)), _mid((NC, SB, 64))),
    )(hc, agg3, Wo[LAYERS - 1])

    grp = _sc_scatter_cols(br3, batch_p, SGP, z64g, ch=112)
    gr = _tc(
        _k_graph_body, (1,), [_full((NC, SGP, 64))],
        jax.ShapeDtypeStruct((SGP, HIDDEN), f32), _full((SGP, HIDDEN)),
    )(grp)

    return (hb[:N_BLOCK], br[:N_BLOCK], gr[:N_GRAPH])
